# baseline (speedup 1.0000x reference)
import functools

import jax
import jax.numpy as jnp
from jax import lax
from jax.experimental import pallas as pl
from jax.experimental.pallas import tpu as pltpu
from jax.experimental.pallas import tpu_sc as plsc

N = 10000
E = 160000
D_IN = 128
H1 = 128
H2 = 32

NC, NS = 2, 16
NW = NC * NS
CHUNK = 128
CPW = 40
E_PAD = NW * CPW * CHUNK
R_PAD = 10112
RPT = R_PAD // NS

BM = 1024
NBUF = 2
NDST = 4


@functools.lru_cache(maxsize=None)
def _make_sc_agg(with_deg):
    mesh = plsc.VectorSubcoreMesh(core_axis_name="c", subcore_axis_name="s",
                                  num_cores=NC, num_subcores=NS)
    out_type = [jax.ShapeDtypeStruct((NC, R_PAD, H1), jnp.float32)]
    scratch = [
        pltpu.VMEM((CPW, CHUNK), jnp.int32),
        pltpu.VMEM((NDST, CHUNK), jnp.int32),
        pltpu.VMEM((NBUF, CHUNK, H1), jnp.float32),
        pltpu.VMEM_SHARED((R_PAD, H1), jnp.float32),
        pltpu.SemaphoreType.DMA((NBUF,)),
        pltpu.SemaphoreType.DMA((NBUF,)),
        pltpu.SemaphoreType.DMA((NDST,)),
    ]
    if with_deg:
        out_type.append(jax.ShapeDtypeStruct((NW, R_PAD), jnp.float32))
        scratch.append(pltpu.VMEM((R_PAD,), jnp.float32))

    @functools.partial(pl.kernel, out_type=out_type, mesh=mesh,
                       scratch_types=scratch,
                       compiler_params=pltpu.CompilerParams(
                           needs_layout_passes=False))
    def sc_agg(y_hbm, src_hbm, dst_hbm, zeros_hbm, zerosd_hbm, *refs):
        if with_deg:
            (out_hbm, deg_hbm, src_v, dst_r, rows_v, acc_sh,
             gsem, ssem, dsem, deg_v) = refs
        else:
            out_hbm, src_v, dst_r, rows_v, acc_sh, gsem, ssem, dsem = refs
        cid = lax.axis_index("c")
        sid = lax.axis_index("s")
        wid = cid * NS + sid
        pltpu.sync_copy(src_hbm.at[wid], src_v)
        pltpu.sync_copy(zeros_hbm.at[pl.ds(sid * RPT, RPT)],
                        acc_sh.at[pl.ds(sid * RPT, RPT)])
        if with_deg:
            pltpu.sync_copy(zerosd_hbm, deg_v)
        plsc.subcore_barrier()

        ones16 = jnp.full((16,), 1.0, jnp.float32)

        def dst_load(s, j):
            pltpu.async_copy(dst_hbm.at[wid].at[j], dst_r.at[s], dsem.at[s])

        def gather(r, j):
            pltpu.async_copy(y_hbm.at[src_v.at[j]], rows_v.at[r], gsem.at[r])

        for s in range(NDST):
            dst_load(s, s)
        for r in range(NBUF):
            gather(r, r)

        def body(g, _):
            for b in range(NDST):
                j = g * NDST + b
                r = b % NBUF
                s = b
                pltpu.make_async_copy(y_hbm.at[src_v.at[j]], rows_v.at[r],
                                      gsem.at[r]).wait()
                pltpu.make_async_copy(dst_hbm.at[wid].at[j], dst_r.at[s],
                                      dsem.at[s]).wait()
                pltpu.async_copy(rows_v.at[r], acc_sh.at[dst_r.at[s]],
                                 ssem.at[r], add=True)
                if with_deg:
                    for t in range(CHUNK // 16):
                        idx = dst_r[s, pl.ds(t * 16, 16)]
                        plsc.addupdate_scatter(deg_v, [idx], ones16)
                pltpu.make_async_copy(rows_v.at[r], acc_sh.at[dst_r.at[s]],
                                      ssem.at[r]).wait()

                @pl.when(j + NDST < CPW)
                def _():
                    dst_load(s, j + NDST)

                @pl.when(j + NBUF < CPW)
                def _():
                    gather(r, j + NBUF)
            return ()

        lax.fori_loop(0, CPW // NDST, body, ())
        plsc.subcore_barrier()
        pltpu.sync_copy(acc_sh.at[pl.ds(sid * RPT, RPT)],
                        out_hbm.at[cid, pl.ds(sid * RPT, RPT)])
        if with_deg:
            pltpu.sync_copy(deg_v, deg_hbm.at[wid])

    return sc_agg


def _agg(y, src3, dst3, zeros, zerosd, with_deg):
    return _make_sc_agg(with_deg)(y, src3, dst3, zeros, zerosd)


def _proj1_body(x_ref, wn_ref, ws_ref, y_ref, s_ref):
    x = x_ref[...]
    y_ref[...] = jnp.dot(x, wn_ref[...], preferred_element_type=jnp.float32)
    s_ref[...] = jnp.dot(x, ws_ref[...], preferred_element_type=jnp.float32)


def _proj1(x, w_neigh, w_self):
    return pl.pallas_call(
        _proj1_body,
        grid=(pl.cdiv(N, BM),),
        in_specs=[
            pl.BlockSpec((BM, D_IN), lambda i: (i, 0)),
            pl.BlockSpec((D_IN, H1), lambda i: (0, 0)),
            pl.BlockSpec((D_IN, H1), lambda i: (0, 0)),
        ],
        out_specs=[
            pl.BlockSpec((BM, H1), lambda i: (i, 0)),
            pl.BlockSpec((BM, H1), lambda i: (i, 0)),
        ],
        out_shape=[
            jax.ShapeDtypeStruct((N, H1), jnp.float32),
            jax.ShapeDtypeStruct((N, H1), jnp.float32),
        ],
    )(x, w_neigh, w_self)


def _layerB_body(s1_ref, p0_ref, p1_ref, deg_ref, b1_ref, ws_ref,
                 h_ref, s2_ref):
    agg = p0_ref[0] + p1_ref[0]
    deg = jnp.sum(deg_ref[...], axis=0)
    hn = agg / jnp.maximum(deg, 1.0)[:, None]
    h = jnp.maximum(s1_ref[...] + hn + b1_ref[...], 0.0)
    h_ref[...] = h
    s2_ref[...] = jnp.dot(h, ws_ref[...], preferred_element_type=jnp.float32)


def _layerB(s1, p, deg, b1, w_self):
    return pl.pallas_call(
        _layerB_body,
        grid=(pl.cdiv(N, BM),),
        in_specs=[
            pl.BlockSpec((BM, H1), lambda i: (i, 0)),
            pl.BlockSpec((1, BM, H1), lambda i: (0, i, 0)),
            pl.BlockSpec((1, BM, H1), lambda i: (1, i, 0)),
            pl.BlockSpec((NW, BM), lambda i: (0, i)),
            pl.BlockSpec((1, H1), lambda i: (0, 0)),
            pl.BlockSpec((H1, H2), lambda i: (0, 0)),
        ],
        out_specs=[
            pl.BlockSpec((BM, H1), lambda i: (i, 0)),
            pl.BlockSpec((BM, H2), lambda i: (i, 0)),
        ],
        out_shape=[
            jax.ShapeDtypeStruct((N, H1), jnp.float32),
            jax.ShapeDtypeStruct((N, H2), jnp.float32),
        ],
    )(s1, p, p, deg, b1, w_self)


def _layerC1_body(s2_ref, q0_ref, q1_ref, deg_ref, b2_ref, wn_ref, z_ref):
    agg = q0_ref[0] + q1_ref[0]
    deg = jnp.sum(deg_ref[...], axis=0)
    hn = agg / jnp.maximum(deg, 1.0)[:, None]
    proj = jnp.dot(hn, wn_ref[...], preferred_element_type=jnp.float32)
    z_ref[...] = jnp.maximum(s2_ref[...] + proj + b2_ref[...], 0.0)


def _layerC1(s2, q, deg, b2, w_neigh):
    return pl.pallas_call(
        _layerC1_body,
        grid=(pl.cdiv(N, BM),),
        in_specs=[
            pl.BlockSpec((BM, H2), lambda i: (i, 0)),
            pl.BlockSpec((1, BM, H1), lambda i: (0, i, 0)),
            pl.BlockSpec((1, BM, H1), lambda i: (1, i, 0)),
            pl.BlockSpec((NW, BM), lambda i: (0, i)),
            pl.BlockSpec((1, H2), lambda i: (0, 0)),
            pl.BlockSpec((H1, H2), lambda i: (0, 0)),
        ],
        out_specs=pl.BlockSpec((BM, H2), lambda i: (i, 0)),
        out_shape=jax.ShapeDtypeStruct((N, H2), jnp.float32),
    )(s2, q, q, deg, b2, w_neigh)


def _dec_body(zi_ref, zj_ref, out_ref):
    prod = lax.dot_general(zi_ref[...], zj_ref[...],
                           (((1,), (1,)), ((), ())),
                           preferred_element_type=jnp.float32)
    out_ref[...] = jax.nn.sigmoid(prod)


DEC_B = 1024


def _decoder(z):
    nb = pl.cdiv(N, DEC_B)
    return pl.pallas_call(
        _dec_body,
        grid=(nb, nb),
        in_specs=[
            pl.BlockSpec((DEC_B, H2), lambda i, j: (i, 0)),
            pl.BlockSpec((DEC_B, H2), lambda i, j: (j, 0)),
        ],
        out_specs=pl.BlockSpec((DEC_B, DEC_B), lambda i, j: (i, j)),
        out_shape=jax.ShapeDtypeStruct((N, N), jnp.float32),
    )(z, z)


def kernel(features, edge_index, W1_self, W1_neigh, b1, W2_self, W2_neigh, b2):
    src = edge_index[0].astype(jnp.int32)
    dst = edge_index[1].astype(jnp.int32)
    pad_dst = N + (jnp.arange(E_PAD - E, dtype=jnp.int32) % (R_PAD - N))
    src3 = jnp.concatenate(
        [src, jnp.zeros((E_PAD - E,), jnp.int32)]).reshape(NW, CPW, CHUNK)
    dst3 = jnp.concatenate([dst, pad_dst]).reshape(NW, CPW, CHUNK)
    zeros2d = jnp.zeros((R_PAD, H1), jnp.float32)
    zeros1d = jnp.zeros((R_PAD,), jnp.float32)

    y1, s1 = _proj1(features, W1_neigh, W1_self)
    p, deg = _agg(y1, src3, dst3, zeros2d, zeros1d, True)
    h, s2 = _layerB(s1, p, deg, b1.reshape(1, H1), W2_self)
    (q,) = _agg(h, src3, dst3, zeros2d, zeros1d, False)
    z = _layerC1(s2, q, deg, b2.reshape(1, H2), W2_neigh)
    return _decoder(z)


# speedup vs baseline: 1.0003x; 1.0003x over previous
import functools

import jax
import jax.numpy as jnp
from jax import lax
from jax.experimental import pallas as pl
from jax.experimental.pallas import tpu as pltpu
from jax.experimental.pallas import tpu_sc as plsc

N = 10000
E = 160000
D_IN = 128
H1 = 128
H2 = 32

NC, NS = 2, 16
NW = NC * NS
CHUNK = 128
CPW = 40
E_PAD = NW * CPW * CHUNK
R_PAD = 10112
RPT = R_PAD // NS

BM = 1024
NBUF = 2
NDST = 4


@functools.lru_cache(maxsize=None)
def _make_sc_agg(with_deg):
    mesh = plsc.VectorSubcoreMesh(core_axis_name="c", subcore_axis_name="s",
                                  num_cores=NC, num_subcores=NS)
    out_type = [jax.ShapeDtypeStruct((NC, R_PAD, H1), jnp.float32)]
    scratch = [
        pltpu.VMEM((CPW, CHUNK), jnp.int32),
        pltpu.VMEM((NDST, CHUNK), jnp.int32),
        pltpu.VMEM((NBUF, CHUNK, H1), jnp.float32),
        pltpu.VMEM_SHARED((R_PAD, H1), jnp.float32),
        pltpu.SemaphoreType.DMA((NBUF,)),
        pltpu.SemaphoreType.DMA((NBUF,)),
        pltpu.SemaphoreType.DMA((NDST,)),
    ]
    if with_deg:
        out_type.append(jax.ShapeDtypeStruct((NW, R_PAD), jnp.float32))
        scratch.append(pltpu.VMEM((R_PAD,), jnp.float32))

    @functools.partial(pl.kernel, out_type=out_type, mesh=mesh,
                       scratch_types=scratch,
                       compiler_params=pltpu.CompilerParams(
                           needs_layout_passes=False))
    def sc_agg(y_hbm, src_hbm, dst_hbm, zeros_hbm, zerosd_hbm, *refs):
        if with_deg:
            (out_hbm, deg_hbm, src_v, dst_r, rows_v, acc_sh,
             gsem, ssem, dsem, deg_v) = refs
        else:
            out_hbm, src_v, dst_r, rows_v, acc_sh, gsem, ssem, dsem = refs
        cid = lax.axis_index("c")
        sid = lax.axis_index("s")
        wid = cid * NS + sid
        with jax.named_scope("sc_zero"):
            pltpu.sync_copy(src_hbm.at[wid], src_v)
            pltpu.sync_copy(zeros_hbm.at[pl.ds(sid * RPT, RPT)],
                            acc_sh.at[pl.ds(sid * RPT, RPT)])
            if with_deg:
                pltpu.sync_copy(zerosd_hbm, deg_v)
            plsc.subcore_barrier()

        ones16 = jnp.full((16,), 1.0, jnp.float32)

        def dst_load(s, j):
            pltpu.async_copy(dst_hbm.at[wid].at[j], dst_r.at[s], dsem.at[s])

        def gather(r, j):
            pltpu.async_copy(y_hbm.at[src_v.at[j]], rows_v.at[r], gsem.at[r])

        for s in range(NDST):
            dst_load(s, s)
        for r in range(NBUF):
            gather(r, r)

        def body(g, _):
            for b in range(NDST):
                j = g * NDST + b
                r = b % NBUF
                s = b
                pltpu.make_async_copy(y_hbm.at[src_v.at[j]], rows_v.at[r],
                                      gsem.at[r]).wait()
                pltpu.make_async_copy(dst_hbm.at[wid].at[j], dst_r.at[s],
                                      dsem.at[s]).wait()
                pltpu.async_copy(rows_v.at[r], acc_sh.at[dst_r.at[s]],
                                 ssem.at[r], add=True)
                if with_deg:
                    for t in range(CHUNK // 16):
                        idx = dst_r[s, pl.ds(t * 16, 16)]
                        plsc.addupdate_scatter(deg_v, [idx], ones16)
                pltpu.make_async_copy(rows_v.at[r], acc_sh.at[dst_r.at[s]],
                                      ssem.at[r]).wait()

                @pl.when(j + NDST < CPW)
                def _():
                    dst_load(s, j + NDST)

                @pl.when(j + NBUF < CPW)
                def _():
                    gather(r, j + NBUF)
            return ()

        with jax.named_scope("sc_mainloop"):
            lax.fori_loop(0, CPW // NDST, body, ())
            plsc.subcore_barrier()
        with jax.named_scope("sc_copyout"):
            pltpu.sync_copy(acc_sh.at[pl.ds(sid * RPT, RPT)],
                            out_hbm.at[cid, pl.ds(sid * RPT, RPT)])
            if with_deg:
                pltpu.sync_copy(deg_v, deg_hbm.at[wid])

    return sc_agg


def _agg(y, src3, dst3, zeros, zerosd, with_deg):
    return _make_sc_agg(with_deg)(y, src3, dst3, zeros, zerosd)


def _proj1_body(x_ref, wn_ref, ws_ref, y_ref, s_ref):
    x = x_ref[...]
    y_ref[...] = jnp.dot(x, wn_ref[...], preferred_element_type=jnp.float32)
    s_ref[...] = jnp.dot(x, ws_ref[...], preferred_element_type=jnp.float32)


def _proj1(x, w_neigh, w_self):
    return pl.pallas_call(
        _proj1_body,
        grid=(pl.cdiv(N, BM),),
        in_specs=[
            pl.BlockSpec((BM, D_IN), lambda i: (i, 0)),
            pl.BlockSpec((D_IN, H1), lambda i: (0, 0)),
            pl.BlockSpec((D_IN, H1), lambda i: (0, 0)),
        ],
        out_specs=[
            pl.BlockSpec((BM, H1), lambda i: (i, 0)),
            pl.BlockSpec((BM, H1), lambda i: (i, 0)),
        ],
        out_shape=[
            jax.ShapeDtypeStruct((N, H1), jnp.float32),
            jax.ShapeDtypeStruct((N, H1), jnp.float32),
        ],
    )(x, w_neigh, w_self)


def _layerB_body(s1_ref, p0_ref, p1_ref, deg_ref, b1_ref, ws_ref,
                 h_ref, s2_ref):
    agg = p0_ref[0] + p1_ref[0]
    deg = jnp.sum(deg_ref[...], axis=0)
    hn = agg / jnp.maximum(deg, 1.0)[:, None]
    h = jnp.maximum(s1_ref[...] + hn + b1_ref[...], 0.0)
    h_ref[...] = h
    s2_ref[...] = jnp.dot(h, ws_ref[...], preferred_element_type=jnp.float32)


def _layerB(s1, p, deg, b1, w_self):
    return pl.pallas_call(
        _layerB_body,
        grid=(pl.cdiv(N, BM),),
        in_specs=[
            pl.BlockSpec((BM, H1), lambda i: (i, 0)),
            pl.BlockSpec((1, BM, H1), lambda i: (0, i, 0)),
            pl.BlockSpec((1, BM, H1), lambda i: (1, i, 0)),
            pl.BlockSpec((NW, BM), lambda i: (0, i)),
            pl.BlockSpec((1, H1), lambda i: (0, 0)),
            pl.BlockSpec((H1, H2), lambda i: (0, 0)),
        ],
        out_specs=[
            pl.BlockSpec((BM, H1), lambda i: (i, 0)),
            pl.BlockSpec((BM, H2), lambda i: (i, 0)),
        ],
        out_shape=[
            jax.ShapeDtypeStruct((N, H1), jnp.float32),
            jax.ShapeDtypeStruct((N, H2), jnp.float32),
        ],
    )(s1, p, p, deg, b1, w_self)


def _layerC1_body(s2_ref, q0_ref, q1_ref, deg_ref, b2_ref, wn_ref, z_ref):
    agg = q0_ref[0] + q1_ref[0]
    deg = jnp.sum(deg_ref[...], axis=0)
    hn = agg / jnp.maximum(deg, 1.0)[:, None]
    proj = jnp.dot(hn, wn_ref[...], preferred_element_type=jnp.float32)
    z_ref[...] = jnp.maximum(s2_ref[...] + proj + b2_ref[...], 0.0)


def _layerC1(s2, q, deg, b2, w_neigh):
    return pl.pallas_call(
        _layerC1_body,
        grid=(pl.cdiv(N, BM),),
        in_specs=[
            pl.BlockSpec((BM, H2), lambda i: (i, 0)),
            pl.BlockSpec((1, BM, H1), lambda i: (0, i, 0)),
            pl.BlockSpec((1, BM, H1), lambda i: (1, i, 0)),
            pl.BlockSpec((NW, BM), lambda i: (0, i)),
            pl.BlockSpec((1, H2), lambda i: (0, 0)),
            pl.BlockSpec((H1, H2), lambda i: (0, 0)),
        ],
        out_specs=pl.BlockSpec((BM, H2), lambda i: (i, 0)),
        out_shape=jax.ShapeDtypeStruct((N, H2), jnp.float32),
    )(s2, q, q, deg, b2, w_neigh)


def _dec_body(zi_ref, zj_ref, out_ref):
    prod = lax.dot_general(zi_ref[...], zj_ref[...],
                           (((1,), (1,)), ((), ())),
                           preferred_element_type=jnp.float32)
    out_ref[...] = jax.nn.sigmoid(prod)


DEC_B = 1024


def _decoder(z):
    nb = pl.cdiv(N, DEC_B)
    return pl.pallas_call(
        _dec_body,
        grid=(nb, nb),
        in_specs=[
            pl.BlockSpec((DEC_B, H2), lambda i, j: (i, 0)),
            pl.BlockSpec((DEC_B, H2), lambda i, j: (j, 0)),
        ],
        out_specs=pl.BlockSpec((DEC_B, DEC_B), lambda i, j: (i, j)),
        out_shape=jax.ShapeDtypeStruct((N, N), jnp.float32),
    )(z, z)


def kernel(features, edge_index, W1_self, W1_neigh, b1, W2_self, W2_neigh, b2):
    src = edge_index[0].astype(jnp.int32)
    dst = edge_index[1].astype(jnp.int32)
    pad_dst = N + (jnp.arange(E_PAD - E, dtype=jnp.int32) % (R_PAD - N))
    src3 = jnp.concatenate(
        [src, jnp.zeros((E_PAD - E,), jnp.int32)]).reshape(NW, CPW, CHUNK)
    dst3 = jnp.concatenate([dst, pad_dst]).reshape(NW, CPW, CHUNK)
    zeros2d = jnp.zeros((R_PAD, H1), jnp.float32)
    zeros1d = jnp.zeros((R_PAD,), jnp.float32)

    y1, s1 = _proj1(features, W1_neigh, W1_self)
    p, deg = _agg(y1, src3, dst3, zeros2d, zeros1d, True)
    h, s2 = _layerB(s1, p, deg, b1.reshape(1, H1), W2_self)
    (q,) = _agg(h, src3, dst3, zeros2d, zeros1d, False)
    z = _layerC1(s2, q, deg, b2.reshape(1, H2), W2_neigh)
    return _decoder(z)


# speedup vs baseline: 1.0005x; 1.0001x over previous
import functools

import jax
import jax.numpy as jnp
from jax import lax
from jax.experimental import pallas as pl
from jax.experimental.pallas import tpu as pltpu
from jax.experimental.pallas import tpu_sc as plsc

N = 10000
E = 160000
D_IN = 128
H1 = 128
H2 = 32

NC, NS = 2, 16
NW = NC * NS
CHUNK = 128
CPW = 40
E_PAD = NW * CPW * CHUNK
R_PAD = 10112
RPT = R_PAD // NS

BM = 1024
NBUF = 2
NDST = 4


@functools.lru_cache(maxsize=None)
def _make_sc_agg(with_deg):
    mesh = plsc.VectorSubcoreMesh(core_axis_name="c", subcore_axis_name="s",
                                  num_cores=NC, num_subcores=NS)
    out_type = [jax.ShapeDtypeStruct((NC, R_PAD, H1), jnp.float32)]
    scratch = [
        pltpu.VMEM((CPW, CHUNK), jnp.int32),
        pltpu.VMEM((NDST, CHUNK), jnp.int32),
        pltpu.VMEM((NBUF, CHUNK, H1), jnp.float32),
        pltpu.VMEM_SHARED((R_PAD, H1), jnp.float32),
        pltpu.SemaphoreType.DMA((NBUF,)),
        pltpu.SemaphoreType.DMA((NBUF,)),
        pltpu.SemaphoreType.DMA((NDST,)),
    ]
    if with_deg:
        out_type.append(jax.ShapeDtypeStruct((NW, R_PAD), jnp.float32))
        scratch.append(pltpu.VMEM((R_PAD,), jnp.float32))

    @functools.partial(pl.kernel, out_type=out_type, mesh=mesh,
                       scratch_types=scratch,
                       compiler_params=pltpu.CompilerParams(
                           needs_layout_passes=False))
    def sc_agg(y_hbm, src_hbm, dst_hbm, zeros_hbm, zerosd_hbm, *refs):
        if with_deg:
            (out_hbm, deg_hbm, src_v, dst_r, rows_v, acc_sh,
             gsem, ssem, dsem, deg_v) = refs
        else:
            out_hbm, src_v, dst_r, rows_v, acc_sh, gsem, ssem, dsem = refs
        cid = lax.axis_index("c")
        sid = lax.axis_index("s")
        wid = cid * NS + sid
        with jax.named_scope("sc_zero"):
            pltpu.sync_copy(src_hbm.at[wid], src_v)
            pltpu.sync_copy(zeros_hbm.at[pl.ds(sid * RPT, RPT)],
                            acc_sh.at[pl.ds(sid * RPT, RPT)])
            if with_deg:
                pltpu.sync_copy(zerosd_hbm, deg_v)
            plsc.subcore_barrier()

        ones16 = jnp.full((16,), 1.0, jnp.float32)

        def dst_load(s, j):
            pltpu.async_copy(dst_hbm.at[wid].at[j], dst_r.at[s], dsem.at[s])

        def gather(r, j):
            pltpu.async_copy(y_hbm.at[src_v.at[j]], rows_v.at[r], gsem.at[r])

        for s in range(NDST):
            dst_load(s, s)
        for r in range(NBUF):
            gather(r, r)

        def body(g, _):
            for b in range(NDST):
                j = g * NDST + b
                r = b % NBUF
                s = b
                pltpu.make_async_copy(y_hbm.at[src_v.at[j]], rows_v.at[r],
                                      gsem.at[r]).wait()
                pltpu.make_async_copy(dst_hbm.at[wid].at[j], dst_r.at[s],
                                      dsem.at[s]).wait()
                pltpu.async_copy(rows_v.at[r], acc_sh.at[dst_r.at[s]],
                                 ssem.at[r], add=True)
                if with_deg:
                    for t in range(CHUNK // 16):
                        idx = dst_r[s, pl.ds(t * 16, 16)]
                        plsc.addupdate_scatter(deg_v, [idx], ones16)
                pltpu.make_async_copy(rows_v.at[r], acc_sh.at[dst_r.at[s]],
                                      ssem.at[r]).wait()

                @pl.when(j + NDST < CPW)
                def _():
                    dst_load(s, j + NDST)

                @pl.when(j + NBUF < CPW)
                def _():
                    gather(r, j + NBUF)
            return ()

        with jax.named_scope("sc_mainloop"):
            lax.fori_loop(0, CPW // NDST, body, ())
        with jax.named_scope("sc_bar"):
            plsc.subcore_barrier()
        with jax.named_scope("sc_copyout"):
            pltpu.sync_copy(acc_sh.at[pl.ds(sid * RPT, RPT)],
                            out_hbm.at[cid, pl.ds(sid * RPT, RPT)])
            if with_deg:
                pltpu.sync_copy(deg_v, deg_hbm.at[wid])

    return sc_agg


def _agg(y, src3, dst3, zeros, zerosd, with_deg):
    return _make_sc_agg(with_deg)(y, src3, dst3, zeros, zerosd)


def _proj1_body(x_ref, wn_ref, ws_ref, y_ref, s_ref):
    x = x_ref[...]
    y_ref[...] = jnp.dot(x, wn_ref[...], preferred_element_type=jnp.float32)
    s_ref[...] = jnp.dot(x, ws_ref[...], preferred_element_type=jnp.float32)


def _proj1(x, w_neigh, w_self):
    return pl.pallas_call(
        _proj1_body,
        grid=(pl.cdiv(N, BM),),
        in_specs=[
            pl.BlockSpec((BM, D_IN), lambda i: (i, 0)),
            pl.BlockSpec((D_IN, H1), lambda i: (0, 0)),
            pl.BlockSpec((D_IN, H1), lambda i: (0, 0)),
        ],
        out_specs=[
            pl.BlockSpec((BM, H1), lambda i: (i, 0)),
            pl.BlockSpec((BM, H1), lambda i: (i, 0)),
        ],
        out_shape=[
            jax.ShapeDtypeStruct((N, H1), jnp.float32),
            jax.ShapeDtypeStruct((N, H1), jnp.float32),
        ],
    )(x, w_neigh, w_self)


def _layerB_body(s1_ref, p0_ref, p1_ref, deg_ref, b1_ref, ws_ref,
                 h_ref, s2_ref):
    agg = p0_ref[0] + p1_ref[0]
    deg = jnp.sum(deg_ref[...], axis=0)
    hn = agg / jnp.maximum(deg, 1.0)[:, None]
    h = jnp.maximum(s1_ref[...] + hn + b1_ref[...], 0.0)
    h_ref[...] = h
    s2_ref[...] = jnp.dot(h, ws_ref[...], preferred_element_type=jnp.float32)


def _layerB(s1, p, deg, b1, w_self):
    return pl.pallas_call(
        _layerB_body,
        grid=(pl.cdiv(N, BM),),
        in_specs=[
            pl.BlockSpec((BM, H1), lambda i: (i, 0)),
            pl.BlockSpec((1, BM, H1), lambda i: (0, i, 0)),
            pl.BlockSpec((1, BM, H1), lambda i: (1, i, 0)),
            pl.BlockSpec((NW, BM), lambda i: (0, i)),
            pl.BlockSpec((1, H1), lambda i: (0, 0)),
            pl.BlockSpec((H1, H2), lambda i: (0, 0)),
        ],
        out_specs=[
            pl.BlockSpec((BM, H1), lambda i: (i, 0)),
            pl.BlockSpec((BM, H2), lambda i: (i, 0)),
        ],
        out_shape=[
            jax.ShapeDtypeStruct((N, H1), jnp.float32),
            jax.ShapeDtypeStruct((N, H2), jnp.float32),
        ],
    )(s1, p, p, deg, b1, w_self)


def _layerC1_body(s2_ref, q0_ref, q1_ref, deg_ref, b2_ref, wn_ref, z_ref):
    agg = q0_ref[0] + q1_ref[0]
    deg = jnp.sum(deg_ref[...], axis=0)
    hn = agg / jnp.maximum(deg, 1.0)[:, None]
    proj = jnp.dot(hn, wn_ref[...], preferred_element_type=jnp.float32)
    z_ref[...] = jnp.maximum(s2_ref[...] + proj + b2_ref[...], 0.0)


def _layerC1(s2, q, deg, b2, w_neigh):
    return pl.pallas_call(
        _layerC1_body,
        grid=(pl.cdiv(N, BM),),
        in_specs=[
            pl.BlockSpec((BM, H2), lambda i: (i, 0)),
            pl.BlockSpec((1, BM, H1), lambda i: (0, i, 0)),
            pl.BlockSpec((1, BM, H1), lambda i: (1, i, 0)),
            pl.BlockSpec((NW, BM), lambda i: (0, i)),
            pl.BlockSpec((1, H2), lambda i: (0, 0)),
            pl.BlockSpec((H1, H2), lambda i: (0, 0)),
        ],
        out_specs=pl.BlockSpec((BM, H2), lambda i: (i, 0)),
        out_shape=jax.ShapeDtypeStruct((N, H2), jnp.float32),
    )(s2, q, q, deg, b2, w_neigh)


def _dec_body(zi_ref, zj_ref, out_ref):
    prod = lax.dot_general(zi_ref[...], zj_ref[...],
                           (((1,), (1,)), ((), ())),
                           preferred_element_type=jnp.float32)
    out_ref[...] = jax.nn.sigmoid(prod)


DEC_B = 1024


def _decoder(z):
    nb = pl.cdiv(N, DEC_B)
    return pl.pallas_call(
        _dec_body,
        grid=(nb, nb),
        in_specs=[
            pl.BlockSpec((DEC_B, H2), lambda i, j: (i, 0)),
            pl.BlockSpec((DEC_B, H2), lambda i, j: (j, 0)),
        ],
        out_specs=pl.BlockSpec((DEC_B, DEC_B), lambda i, j: (i, j)),
        out_shape=jax.ShapeDtypeStruct((N, N), jnp.float32),
    )(z, z)


def kernel(features, edge_index, W1_self, W1_neigh, b1, W2_self, W2_neigh, b2):
    src = edge_index[0].astype(jnp.int32)
    dst = edge_index[1].astype(jnp.int32)
    pad_dst = N + (jnp.arange(E_PAD - E, dtype=jnp.int32) % (R_PAD - N))
    src3 = jnp.concatenate(
        [src, jnp.zeros((E_PAD - E,), jnp.int32)]).reshape(NW, CPW, CHUNK)
    dst3 = jnp.concatenate([dst, pad_dst]).reshape(NW, CPW, CHUNK)
    zeros2d = jnp.zeros((R_PAD, H1), jnp.float32)
    zeros1d = jnp.zeros((R_PAD,), jnp.float32)

    y1, s1 = _proj1(features, W1_neigh, W1_self)
    p, deg = _agg(y1, src3, dst3, zeros2d, zeros1d, True)
    h, s2 = _layerB(s1, p, deg, b1.reshape(1, H1), W2_self)
    (q,) = _agg(h, src3, dst3, zeros2d, zeros1d, False)
    z = _layerC1(s2, q, deg, b2.reshape(1, H2), W2_neigh)
    return _decoder(z)


# speedup vs baseline: 2.1043x; 2.1034x over previous
import functools

import jax
import jax.numpy as jnp
from jax import lax
from jax.experimental import pallas as pl
from jax.experimental.pallas import tpu as pltpu
from jax.experimental.pallas import tpu_sc as plsc

N = 10000
E = 160000
D_IN = 128
H1 = 128
H2 = 32

NC, NS = 2, 16
NW = NC * NS
CHUNK = 128
CPW = 40
E_PAD = NW * CPW * CHUNK
R_PAD = 10112
RPT = R_PAD // NS

BM = 1024
NBUF = 2
NDST = 4


@functools.lru_cache(maxsize=None)
def _make_sc_agg(with_deg):
    mesh = plsc.VectorSubcoreMesh(core_axis_name="c", subcore_axis_name="s",
                                  num_cores=NC, num_subcores=NS)
    out_type = [jax.ShapeDtypeStruct((NC, R_PAD, H1), jnp.float32)]
    scratch = [
        pltpu.VMEM((CPW, CHUNK), jnp.int32),
        pltpu.VMEM((NDST, CHUNK), jnp.int32),
        pltpu.VMEM((NBUF, CHUNK, H1), jnp.float32),
        pltpu.VMEM_SHARED((R_PAD, H1), jnp.float32),
        pltpu.SemaphoreType.DMA((NBUF,)),
        pltpu.SemaphoreType.DMA((NBUF,)),
        pltpu.SemaphoreType.DMA((NDST,)),
    ]
    if with_deg:
        out_type.append(jax.ShapeDtypeStruct((NW, R_PAD), jnp.float32))
        scratch.append(pltpu.VMEM((R_PAD,), jnp.float32))

    @functools.partial(pl.kernel, out_type=out_type, mesh=mesh,
                       scratch_types=scratch,
                       compiler_params=pltpu.CompilerParams(
                           needs_layout_passes=False))
    def sc_agg(y_hbm, src_hbm, dst_hbm, zeros_hbm, zerosd_hbm, *refs):
        if with_deg:
            (out_hbm, deg_hbm, src_v, dst_r, rows_v, acc_sh,
             gsem, ssem, dsem, deg_v) = refs
        else:
            out_hbm, src_v, dst_r, rows_v, acc_sh, gsem, ssem, dsem = refs
        cid = lax.axis_index("c")
        sid = lax.axis_index("s")
        wid = cid * NS + sid
        with jax.named_scope("sc_zero"):
            pltpu.sync_copy(src_hbm.at[wid], src_v)
            pltpu.sync_copy(zeros_hbm.at[pl.ds(sid * RPT, RPT)],
                            acc_sh.at[pl.ds(sid * RPT, RPT)])
            if with_deg:
                pltpu.sync_copy(zerosd_hbm, deg_v)
            plsc.subcore_barrier()

        ones16 = jnp.full((16,), 1.0, jnp.float32)

        def dst_load(s, j):
            pltpu.async_copy(dst_hbm.at[wid].at[j], dst_r.at[s], dsem.at[s])

        def gather(r, j):
            pltpu.async_copy(y_hbm.at[src_v.at[j]], rows_v.at[r], gsem.at[r])

        for s in range(NDST):
            dst_load(s, s)
        for r in range(NBUF):
            gather(r, r)

        def body(g, _):
            for b in range(NDST):
                j = g * NDST + b
                r = b % NBUF
                s = b
                pltpu.make_async_copy(y_hbm.at[src_v.at[j]], rows_v.at[r],
                                      gsem.at[r]).wait()
                pltpu.make_async_copy(dst_hbm.at[wid].at[j], dst_r.at[s],
                                      dsem.at[s]).wait()
                pltpu.async_copy(rows_v.at[r], acc_sh.at[dst_r.at[s]],
                                 ssem.at[r], add=True)
                if with_deg:
                    for t in range(CHUNK // 16):
                        idx = dst_r[s, pl.ds(t * 16, 16)]
                        plsc.addupdate_scatter(deg_v, [idx], ones16)
                pltpu.make_async_copy(rows_v.at[r], acc_sh.at[dst_r.at[s]],
                                      ssem.at[r]).wait()

                @pl.when(j + NDST < CPW)
                def _():
                    dst_load(s, j + NDST)

                @pl.when(j + NBUF < CPW)
                def _():
                    gather(r, j + NBUF)
            return ()

        with jax.named_scope("sc_mainloop"):
            lax.fori_loop(0, CPW // NDST, body, ())
        with jax.named_scope("sc_bar"):
            plsc.subcore_barrier()
        with jax.named_scope("sc_copyout"):
            pltpu.sync_copy(acc_sh.at[pl.ds(sid * RPT, RPT)],
                            out_hbm.at[cid, pl.ds(sid * RPT, RPT)])
            if with_deg:
                pltpu.sync_copy(deg_v, deg_hbm.at[wid])

    return sc_agg


def _agg(y, src3, dst3, zeros, zerosd, with_deg):
    return _make_sc_agg(with_deg)(y, src3, dst3, zeros, zerosd)


def _proj1_body(x_ref, wn_ref, ws_ref, y_ref, s_ref):
    x = x_ref[...]
    y_ref[...] = jnp.dot(x, wn_ref[...], preferred_element_type=jnp.float32)
    s_ref[...] = jnp.dot(x, ws_ref[...], preferred_element_type=jnp.float32)


def _proj1(x, w_neigh, w_self):
    return pl.pallas_call(
        _proj1_body,
        grid=(pl.cdiv(N, BM),),
        in_specs=[
            pl.BlockSpec((BM, D_IN), lambda i: (i, 0)),
            pl.BlockSpec((D_IN, H1), lambda i: (0, 0)),
            pl.BlockSpec((D_IN, H1), lambda i: (0, 0)),
        ],
        out_specs=[
            pl.BlockSpec((BM, H1), lambda i: (i, 0)),
            pl.BlockSpec((BM, H1), lambda i: (i, 0)),
        ],
        out_shape=[
            jax.ShapeDtypeStruct((N, H1), jnp.float32),
            jax.ShapeDtypeStruct((N, H1), jnp.float32),
        ],
    )(x, w_neigh, w_self)


def _layerB_body(s1_ref, p0_ref, p1_ref, deg_ref, b1_ref, ws_ref,
                 h_ref, s2_ref):
    agg = p0_ref[0] + p1_ref[0]
    deg = jnp.sum(deg_ref[...], axis=0)
    hn = agg / jnp.maximum(deg, 1.0)[:, None]
    h = jnp.maximum(s1_ref[...] + hn + b1_ref[...], 0.0)
    h_ref[...] = h
    s2_ref[...] = jnp.dot(h, ws_ref[...], preferred_element_type=jnp.float32)


def _layerB(s1, p, deg, b1, w_self):
    return pl.pallas_call(
        _layerB_body,
        grid=(pl.cdiv(N, BM),),
        in_specs=[
            pl.BlockSpec((BM, H1), lambda i: (i, 0)),
            pl.BlockSpec((1, BM, H1), lambda i: (0, i, 0)),
            pl.BlockSpec((1, BM, H1), lambda i: (1, i, 0)),
            pl.BlockSpec((NW, BM), lambda i: (0, i)),
            pl.BlockSpec((1, H1), lambda i: (0, 0)),
            pl.BlockSpec((H1, H2), lambda i: (0, 0)),
        ],
        out_specs=[
            pl.BlockSpec((BM, H1), lambda i: (i, 0)),
            pl.BlockSpec((BM, H2), lambda i: (i, 0)),
        ],
        out_shape=[
            jax.ShapeDtypeStruct((N, H1), jnp.float32),
            jax.ShapeDtypeStruct((N, H2), jnp.float32),
        ],
    )(s1, p, p, deg, b1, w_self)


def _layerC1_body(s2_ref, q0_ref, q1_ref, deg_ref, b2_ref, wn_ref, z_ref):
    agg = q0_ref[0] + q1_ref[0]
    deg = jnp.sum(deg_ref[...], axis=0)
    hn = agg / jnp.maximum(deg, 1.0)[:, None]
    proj = jnp.dot(hn, wn_ref[...], preferred_element_type=jnp.float32)
    z_ref[...] = jnp.maximum(s2_ref[...] + proj + b2_ref[...], 0.0)


def _layerC1(s2, q, deg, b2, w_neigh):
    return pl.pallas_call(
        _layerC1_body,
        grid=(pl.cdiv(N, BM),),
        in_specs=[
            pl.BlockSpec((BM, H2), lambda i: (i, 0)),
            pl.BlockSpec((1, BM, H1), lambda i: (0, i, 0)),
            pl.BlockSpec((1, BM, H1), lambda i: (1, i, 0)),
            pl.BlockSpec((NW, BM), lambda i: (0, i)),
            pl.BlockSpec((1, H2), lambda i: (0, 0)),
            pl.BlockSpec((H1, H2), lambda i: (0, 0)),
        ],
        out_specs=pl.BlockSpec((BM, H2), lambda i: (i, 0)),
        out_shape=jax.ShapeDtypeStruct((N, H2), jnp.float32),
    )(s2, q, q, deg, b2, w_neigh)


def _dec_body(zi_ref, zj_ref, out_ref):
    prod = lax.dot_general(zi_ref[...], zj_ref[...],
                           (((1,), (1,)), ((), ())),
                           preferred_element_type=jnp.float32)
    out_ref[...] = jax.nn.sigmoid(prod)


DEC_B = 1024


def _decoder(z):
    nb = pl.cdiv(N, DEC_B)
    return pl.pallas_call(
        _dec_body,
        grid=(nb, nb),
        in_specs=[
            pl.BlockSpec((DEC_B, H2), lambda i, j: (i, 0)),
            pl.BlockSpec((DEC_B, H2), lambda i, j: (j, 0)),
        ],
        out_specs=pl.BlockSpec((DEC_B, DEC_B), lambda i, j: (i, j)),
        out_shape=jax.ShapeDtypeStruct((N, N), jnp.float32),
    )(z, z)


def kernel(features, edge_index, W1_self, W1_neigh, b1, W2_self, W2_neigh, b2):
    src = edge_index[0].astype(jnp.int32)
    dst = edge_index[1].astype(jnp.int32)
    pad_ar = jnp.arange(E_PAD - E, dtype=jnp.int32)
    pad_dst = N + pad_ar % (R_PAD - N)
    pad_src = pad_ar % N
    src3 = jnp.concatenate([src, pad_src]).reshape(NW, CPW, CHUNK)
    dst3 = jnp.concatenate([dst, pad_dst]).reshape(NW, CPW, CHUNK)
    zeros2d = jnp.zeros((R_PAD, H1), jnp.float32)
    zeros1d = jnp.zeros((R_PAD,), jnp.float32)

    y1, s1 = _proj1(features, W1_neigh, W1_self)
    p, deg = _agg(y1, src3, dst3, zeros2d, zeros1d, True)
    h, s2 = _layerB(s1, p, deg, b1.reshape(1, H1), W2_self)
    (q,) = _agg(h, src3, dst3, zeros2d, zeros1d, False)
    z = _layerC1(s2, q, deg, b2.reshape(1, H2), W2_neigh)
    return _decoder(z)


# speedup vs baseline: 2.2772x; 1.0821x over previous
import functools

import jax
import jax.numpy as jnp
from jax import lax
from jax.experimental import pallas as pl
from jax.experimental.pallas import tpu as pltpu
from jax.experimental.pallas import tpu_sc as plsc

N = 10000
E = 160000
D_IN = 128
H1 = 128
H2 = 32

NC, NS = 2, 16
NW = NC * NS
CHUNK = 128
CPW = 40
E_PAD = NW * CPW * CHUNK
R_PAD = 10112
RPT = R_PAD // NS

BM = 1024
NBUF = 2
NDST = 4


@functools.lru_cache(maxsize=None)
def _make_sc_agg(with_deg):
    mesh = plsc.VectorSubcoreMesh(core_axis_name="c", subcore_axis_name="s",
                                  num_cores=NC, num_subcores=NS)
    out_type = [jax.ShapeDtypeStruct((NC, R_PAD, H1), jnp.float32)]
    scratch = [
        pltpu.VMEM((CPW, CHUNK), jnp.int32),
        pltpu.VMEM((NDST, CHUNK), jnp.int32),
        pltpu.VMEM((NBUF, CHUNK, H1), jnp.float32),
        pltpu.VMEM_SHARED((R_PAD, H1), jnp.float32),
        pltpu.SemaphoreType.DMA((NBUF,)),
        pltpu.SemaphoreType.DMA((NBUF,)),
        pltpu.SemaphoreType.DMA((NDST,)),
    ]
    if with_deg:
        out_type.append(jax.ShapeDtypeStruct((NW, R_PAD), jnp.float32))
        scratch.append(pltpu.VMEM((R_PAD,), jnp.float32))

    @functools.partial(pl.kernel, out_type=out_type, mesh=mesh,
                       scratch_types=scratch,
                       compiler_params=pltpu.CompilerParams(
                           needs_layout_passes=False))
    def sc_agg(y_hbm, src_hbm, dst_hbm, zeros_hbm, zerosd_hbm, *refs):
        if with_deg:
            (out_hbm, deg_hbm, src_v, dst_r, rows_v, acc_sh,
             gsem, ssem, dsem, deg_v) = refs
        else:
            out_hbm, src_v, dst_r, rows_v, acc_sh, gsem, ssem, dsem = refs
        cid = lax.axis_index("c")
        sid = lax.axis_index("s")
        wid = cid * NS + sid
        with jax.named_scope("sc_zero"):
            pltpu.sync_copy(src_hbm.at[wid], src_v)
            pltpu.sync_copy(zeros_hbm.at[pl.ds(sid * RPT, RPT)],
                            acc_sh.at[pl.ds(sid * RPT, RPT)])
            if with_deg:
                pltpu.sync_copy(zerosd_hbm, deg_v)
            plsc.subcore_barrier()

        ones16 = jnp.full((16,), 1.0, jnp.float32)

        def dst_load(s, j):
            pltpu.async_copy(dst_hbm.at[wid].at[j], dst_r.at[s], dsem.at[s])

        def gather(r, j):
            pltpu.async_copy(y_hbm.at[src_v.at[j]], rows_v.at[r], gsem.at[r])

        for s in range(NDST):
            dst_load(s, s)
        for r in range(NBUF):
            gather(r, r)

        def body(g, _):
            for b in range(NDST):
                j = g * NDST + b
                r = b % NBUF
                s = b
                pltpu.make_async_copy(y_hbm.at[src_v.at[j]], rows_v.at[r],
                                      gsem.at[r]).wait()
                pltpu.make_async_copy(dst_hbm.at[wid].at[j], dst_r.at[s],
                                      dsem.at[s]).wait()
                pltpu.async_copy(rows_v.at[r], acc_sh.at[dst_r.at[s]],
                                 ssem.at[r], add=True)
                if with_deg:
                    for t in range(CHUNK // 16):
                        idx = dst_r[s, pl.ds(t * 16, 16)]
                        plsc.addupdate_scatter(deg_v, [idx], ones16)
                pltpu.make_async_copy(rows_v.at[r], acc_sh.at[dst_r.at[s]],
                                      ssem.at[r]).wait()

                @pl.when(j + NDST < CPW)
                def _():
                    dst_load(s, j + NDST)

                @pl.when(j + NBUF < CPW)
                def _():
                    gather(r, j + NBUF)
            return ()

        with jax.named_scope("sc_mainloop"):
            lax.fori_loop(0, CPW // NDST, body, ())
        with jax.named_scope("sc_bar"):
            plsc.subcore_barrier()
        with jax.named_scope("sc_copyout"):
            pltpu.sync_copy(acc_sh.at[pl.ds(sid * RPT, RPT)],
                            out_hbm.at[cid, pl.ds(sid * RPT, RPT)])
            if with_deg:
                pltpu.sync_copy(deg_v, deg_hbm.at[wid])

    return sc_agg


def _agg(y, src3, dst3, zeros, zerosd, with_deg):
    return _make_sc_agg(with_deg)(y, src3, dst3, zeros, zerosd)


def _proj1_body(x_ref, wn_ref, ws_ref, y_ref, s_ref):
    x = x_ref[...]
    y_ref[...] = jnp.dot(x, wn_ref[...], preferred_element_type=jnp.float32)
    s_ref[...] = jnp.dot(x, ws_ref[...], preferred_element_type=jnp.float32)


def _proj1(x, w_neigh, w_self):
    return pl.pallas_call(
        _proj1_body,
        grid=(pl.cdiv(N, BM),),
        in_specs=[
            pl.BlockSpec((BM, D_IN), lambda i: (i, 0)),
            pl.BlockSpec((D_IN, H1), lambda i: (0, 0)),
            pl.BlockSpec((D_IN, H1), lambda i: (0, 0)),
        ],
        out_specs=[
            pl.BlockSpec((BM, H1), lambda i: (i, 0)),
            pl.BlockSpec((BM, H1), lambda i: (i, 0)),
        ],
        out_shape=[
            jax.ShapeDtypeStruct((N, H1), jnp.float32),
            jax.ShapeDtypeStruct((N, H1), jnp.float32),
        ],
    )(x, w_neigh, w_self)


def _layerB_body(s1_ref, p0_ref, p1_ref, deg_ref, b1_ref, ws_ref,
                 h_ref, s2_ref):
    agg = p0_ref[0] + p1_ref[0]
    deg = jnp.sum(deg_ref[...], axis=0)
    hn = agg / jnp.maximum(deg, 1.0)[:, None]
    h = jnp.maximum(s1_ref[...] + hn + b1_ref[...], 0.0)
    h_ref[...] = h
    s2_ref[...] = jnp.dot(h, ws_ref[...], preferred_element_type=jnp.float32)


def _layerB(s1, p, deg, b1, w_self):
    return pl.pallas_call(
        _layerB_body,
        grid=(pl.cdiv(N, BM),),
        in_specs=[
            pl.BlockSpec((BM, H1), lambda i: (i, 0)),
            pl.BlockSpec((1, BM, H1), lambda i: (0, i, 0)),
            pl.BlockSpec((1, BM, H1), lambda i: (1, i, 0)),
            pl.BlockSpec((NW, BM), lambda i: (0, i)),
            pl.BlockSpec((1, H1), lambda i: (0, 0)),
            pl.BlockSpec((H1, H2), lambda i: (0, 0)),
        ],
        out_specs=[
            pl.BlockSpec((BM, H1), lambda i: (i, 0)),
            pl.BlockSpec((BM, H2), lambda i: (i, 0)),
        ],
        out_shape=[
            jax.ShapeDtypeStruct((N, H1), jnp.float32),
            jax.ShapeDtypeStruct((N, H2), jnp.float32),
        ],
    )(s1, p, p, deg, b1, w_self)


def _layerC1_body(s2_ref, q0_ref, q1_ref, deg_ref, b2_ref, wn_ref, z_ref):
    agg = q0_ref[0] + q1_ref[0]
    deg = jnp.sum(deg_ref[...], axis=0)
    hn = agg / jnp.maximum(deg, 1.0)[:, None]
    proj = jnp.dot(hn, wn_ref[...], preferred_element_type=jnp.float32)
    z_ref[...] = jnp.maximum(s2_ref[...] + proj + b2_ref[...], 0.0)


def _layerC1(s2, q, deg, b2, w_neigh):
    return pl.pallas_call(
        _layerC1_body,
        grid=(pl.cdiv(N, BM),),
        in_specs=[
            pl.BlockSpec((BM, H2), lambda i: (i, 0)),
            pl.BlockSpec((1, BM, H1), lambda i: (0, i, 0)),
            pl.BlockSpec((1, BM, H1), lambda i: (1, i, 0)),
            pl.BlockSpec((NW, BM), lambda i: (0, i)),
            pl.BlockSpec((1, H2), lambda i: (0, 0)),
            pl.BlockSpec((H1, H2), lambda i: (0, 0)),
        ],
        out_specs=pl.BlockSpec((BM, H2), lambda i: (i, 0)),
        out_shape=jax.ShapeDtypeStruct((N, H2), jnp.float32),
    )(s2, q, q, deg, b2, w_neigh)


def _dec_body(zi_ref, zj_ref, out_ref):
    prod = lax.dot_general(zi_ref[...], zj_ref[...],
                           (((1,), (1,)), ((), ())),
                           preferred_element_type=jnp.float32)
    out_ref[...] = jax.nn.sigmoid(prod)


DEC_BM = 1024
DEC_BN = 2048


def _decoder(z):
    return pl.pallas_call(
        _dec_body,
        grid=(pl.cdiv(N, DEC_BM), pl.cdiv(N, DEC_BN)),
        in_specs=[
            pl.BlockSpec((DEC_BM, H2), lambda i, j: (i, 0)),
            pl.BlockSpec((DEC_BN, H2), lambda i, j: (j, 0)),
        ],
        out_specs=pl.BlockSpec((DEC_BM, DEC_BN), lambda i, j: (i, j)),
        out_shape=jax.ShapeDtypeStruct((N, N), jnp.float32),
    )(z, z)


def kernel(features, edge_index, W1_self, W1_neigh, b1, W2_self, W2_neigh, b2):
    src = edge_index[0].astype(jnp.int32)
    dst = edge_index[1].astype(jnp.int32)
    pad_ar = jnp.arange(E_PAD - E, dtype=jnp.int32)
    pad_dst = N + pad_ar % (R_PAD - N)
    pad_src = pad_ar % N
    src3 = jnp.concatenate([src, pad_src]).reshape(NW, CPW, CHUNK)
    dst3 = jnp.concatenate([dst, pad_dst]).reshape(NW, CPW, CHUNK)
    zeros2d = jnp.zeros((R_PAD, H1), jnp.float32)
    zeros1d = jnp.zeros((R_PAD,), jnp.float32)

    y1, s1 = _proj1(features, W1_neigh, W1_self)
    p, deg = _agg(y1, src3, dst3, zeros2d, zeros1d, True)
    h, s2 = _layerB(s1, p, deg, b1.reshape(1, H1), W2_self)
    (q,) = _agg(h, src3, dst3, zeros2d, zeros1d, False)
    z = _layerC1(s2, q, deg, b2.reshape(1, H2), W2_neigh)
    return _decoder(z)


# speedup vs baseline: 2.3795x; 1.0449x over previous
import functools

import jax
import jax.numpy as jnp
from jax import lax
from jax.experimental import pallas as pl
from jax.experimental.pallas import tpu as pltpu
from jax.experimental.pallas import tpu_sc as plsc

N = 10000
E = 160000
D_IN = 128
H1 = 128
H2 = 32

NC, NS = 2, 16
NW = NC * NS
CHUNK = 128
CPW = 40
E_PAD = NW * CPW * CHUNK
R_PAD = 10112
RPT = R_PAD // NS

BM = 1024
NBUF = 2
NDST = 4


@functools.lru_cache(maxsize=None)
def _make_sc_agg(with_deg):
    mesh = plsc.VectorSubcoreMesh(core_axis_name="c", subcore_axis_name="s",
                                  num_cores=NC, num_subcores=NS)
    out_type = [jax.ShapeDtypeStruct((NC, R_PAD, H1), jnp.float32)]
    scratch = [
        pltpu.VMEM((CPW, CHUNK), jnp.int32),
        pltpu.VMEM((NDST, CHUNK), jnp.int32),
        pltpu.VMEM((NBUF, CHUNK, H1), jnp.float32),
        pltpu.VMEM_SHARED((R_PAD, H1), jnp.float32),
        pltpu.SemaphoreType.DMA((NBUF,)),
        pltpu.SemaphoreType.DMA((NBUF,)),
        pltpu.SemaphoreType.DMA((NDST,)),
    ]
    if with_deg:
        out_type.append(jax.ShapeDtypeStruct((NW, R_PAD), jnp.float32))
        scratch.append(pltpu.VMEM((R_PAD,), jnp.float32))

    @functools.partial(pl.kernel, out_type=out_type, mesh=mesh,
                       scratch_types=scratch,
                       compiler_params=pltpu.CompilerParams(
                           needs_layout_passes=False))
    def sc_agg(y_hbm, src_hbm, dst_hbm, zeros_hbm, zerosd_hbm, *refs):
        if with_deg:
            (out_hbm, deg_hbm, src_v, dst_r, rows_v, acc_sh,
             gsem, ssem, dsem, deg_v) = refs
        else:
            out_hbm, src_v, dst_r, rows_v, acc_sh, gsem, ssem, dsem = refs
        cid = lax.axis_index("c")
        sid = lax.axis_index("s")
        wid = cid * NS + sid
        with jax.named_scope("sc_zero"):
            pltpu.sync_copy(src_hbm.at[wid], src_v)
            pltpu.sync_copy(zeros_hbm.at[pl.ds(sid * RPT, RPT)],
                            acc_sh.at[pl.ds(sid * RPT, RPT)])
            if with_deg:
                pltpu.sync_copy(zerosd_hbm, deg_v)
            plsc.subcore_barrier()

        ones16 = jnp.full((16,), 1.0, jnp.float32)

        def dst_load(s, j):
            pltpu.async_copy(dst_hbm.at[wid].at[j], dst_r.at[s], dsem.at[s])

        def gather(r, j):
            pltpu.async_copy(y_hbm.at[src_v.at[j]], rows_v.at[r], gsem.at[r])

        for s in range(NDST):
            dst_load(s, s)
        for r in range(NBUF):
            gather(r, r)

        def body(g, _):
            for b in range(NDST):
                j = g * NDST + b
                r = b % NBUF
                s = b
                pltpu.make_async_copy(y_hbm.at[src_v.at[j]], rows_v.at[r],
                                      gsem.at[r]).wait()
                pltpu.make_async_copy(dst_hbm.at[wid].at[j], dst_r.at[s],
                                      dsem.at[s]).wait()
                pltpu.async_copy(rows_v.at[r], acc_sh.at[dst_r.at[s]],
                                 ssem.at[r], add=True)
                if with_deg:
                    for t in range(CHUNK // 16):
                        idx = dst_r[s, pl.ds(t * 16, 16)]
                        plsc.addupdate_scatter(deg_v, [idx], ones16)
                pltpu.make_async_copy(rows_v.at[r], acc_sh.at[dst_r.at[s]],
                                      ssem.at[r]).wait()

                @pl.when(j + NDST < CPW)
                def _():
                    dst_load(s, j + NDST)

                @pl.when(j + NBUF < CPW)
                def _():
                    gather(r, j + NBUF)
            return ()

        with jax.named_scope("sc_mainloop"):
            lax.fori_loop(0, CPW // NDST, body, ())
        with jax.named_scope("sc_bar"):
            plsc.subcore_barrier()
        with jax.named_scope("sc_copyout"):
            pltpu.sync_copy(acc_sh.at[pl.ds(sid * RPT, RPT)],
                            out_hbm.at[cid, pl.ds(sid * RPT, RPT)])
            if with_deg:
                pltpu.sync_copy(deg_v, deg_hbm.at[wid])

    return sc_agg


def _agg(y, src3, dst3, zeros, zerosd, with_deg):
    return _make_sc_agg(with_deg)(y, src3, dst3, zeros, zerosd)


def _proj1_body(x_ref, wn_ref, ws_ref, y_ref, s_ref):
    x = x_ref[...]
    y_ref[...] = jnp.dot(x, wn_ref[...], preferred_element_type=jnp.float32)
    s_ref[...] = jnp.dot(x, ws_ref[...], preferred_element_type=jnp.float32)


def _proj1(x, w_neigh, w_self):
    return pl.pallas_call(
        _proj1_body,
        grid=(pl.cdiv(N, BM),),
        in_specs=[
            pl.BlockSpec((BM, D_IN), lambda i: (i, 0)),
            pl.BlockSpec((D_IN, H1), lambda i: (0, 0)),
            pl.BlockSpec((D_IN, H1), lambda i: (0, 0)),
        ],
        out_specs=[
            pl.BlockSpec((BM, H1), lambda i: (i, 0)),
            pl.BlockSpec((BM, H1), lambda i: (i, 0)),
        ],
        out_shape=[
            jax.ShapeDtypeStruct((N, H1), jnp.float32),
            jax.ShapeDtypeStruct((N, H1), jnp.float32),
        ],
    )(x, w_neigh, w_self)


def _layerB_body(s1_ref, p0_ref, p1_ref, deg_ref, b1_ref, ws_ref,
                 h_ref, s2_ref):
    agg = p0_ref[0] + p1_ref[0]
    deg = jnp.sum(deg_ref[...], axis=0)
    hn = agg / jnp.maximum(deg, 1.0)[:, None]
    h = jnp.maximum(s1_ref[...] + hn + b1_ref[...], 0.0)
    h_ref[...] = h
    s2_ref[...] = jnp.dot(h, ws_ref[...], preferred_element_type=jnp.float32)


def _layerB(s1, p, deg, b1, w_self):
    return pl.pallas_call(
        _layerB_body,
        grid=(pl.cdiv(N, BM),),
        in_specs=[
            pl.BlockSpec((BM, H1), lambda i: (i, 0)),
            pl.BlockSpec((1, BM, H1), lambda i: (0, i, 0)),
            pl.BlockSpec((1, BM, H1), lambda i: (1, i, 0)),
            pl.BlockSpec((NW, BM), lambda i: (0, i)),
            pl.BlockSpec((1, H1), lambda i: (0, 0)),
            pl.BlockSpec((H1, H2), lambda i: (0, 0)),
        ],
        out_specs=[
            pl.BlockSpec((BM, H1), lambda i: (i, 0)),
            pl.BlockSpec((BM, H2), lambda i: (i, 0)),
        ],
        out_shape=[
            jax.ShapeDtypeStruct((N, H1), jnp.float32),
            jax.ShapeDtypeStruct((N, H2), jnp.float32),
        ],
    )(s1, p, p, deg, b1, w_self)


def _layerC1_body(s2_ref, q0_ref, q1_ref, deg_ref, b2_ref, wn_ref, z_ref):
    agg = q0_ref[0] + q1_ref[0]
    deg = jnp.sum(deg_ref[...], axis=0)
    hn = agg / jnp.maximum(deg, 1.0)[:, None]
    proj = jnp.dot(hn, wn_ref[...], preferred_element_type=jnp.float32)
    z_ref[...] = jnp.maximum(s2_ref[...] + proj + b2_ref[...], 0.0)


def _layerC1(s2, q, deg, b2, w_neigh):
    return pl.pallas_call(
        _layerC1_body,
        grid=(pl.cdiv(N, BM),),
        in_specs=[
            pl.BlockSpec((BM, H2), lambda i: (i, 0)),
            pl.BlockSpec((1, BM, H1), lambda i: (0, i, 0)),
            pl.BlockSpec((1, BM, H1), lambda i: (1, i, 0)),
            pl.BlockSpec((NW, BM), lambda i: (0, i)),
            pl.BlockSpec((1, H2), lambda i: (0, 0)),
            pl.BlockSpec((H1, H2), lambda i: (0, 0)),
        ],
        out_specs=pl.BlockSpec((BM, H2), lambda i: (i, 0)),
        out_shape=jax.ShapeDtypeStruct((N, H2), jnp.float32),
    )(s2, q, q, deg, b2, w_neigh)


def _dec_body(zi_ref, zj_ref, out_ref):
    prod = lax.dot_general(zi_ref[...], zj_ref[...],
                           (((1,), (1,)), ((), ())),
                           preferred_element_type=jnp.float32)
    out_ref[...] = jax.nn.sigmoid(prod)


DEC_BM = 2048
DEC_BN = 2048


def _decoder(z):
    return pl.pallas_call(
        _dec_body,
        grid=(pl.cdiv(N, DEC_BM), pl.cdiv(N, DEC_BN)),
        in_specs=[
            pl.BlockSpec((DEC_BM, H2), lambda i, j: (i, 0)),
            pl.BlockSpec((DEC_BN, H2), lambda i, j: (j, 0)),
        ],
        out_specs=pl.BlockSpec((DEC_BM, DEC_BN), lambda i, j: (i, j)),
        out_shape=jax.ShapeDtypeStruct((N, N), jnp.float32),
    )(z, z)


def kernel(features, edge_index, W1_self, W1_neigh, b1, W2_self, W2_neigh, b2):
    src = edge_index[0].astype(jnp.int32)
    dst = edge_index[1].astype(jnp.int32)
    pad_ar = jnp.arange(E_PAD - E, dtype=jnp.int32)
    pad_dst = N + pad_ar % (R_PAD - N)
    pad_src = pad_ar % N
    src3 = jnp.concatenate([src, pad_src]).reshape(NW, CPW, CHUNK)
    dst3 = jnp.concatenate([dst, pad_dst]).reshape(NW, CPW, CHUNK)
    zeros2d = jnp.zeros((R_PAD, H1), jnp.float32)
    zeros1d = jnp.zeros((R_PAD,), jnp.float32)

    y1, s1 = _proj1(features, W1_neigh, W1_self)
    p, deg = _agg(y1, src3, dst3, zeros2d, zeros1d, True)
    h, s2 = _layerB(s1, p, deg, b1.reshape(1, H1), W2_self)
    (q,) = _agg(h, src3, dst3, zeros2d, zeros1d, False)
    z = _layerC1(s2, q, deg, b2.reshape(1, H2), W2_neigh)
    return _decoder(z)


# speedup vs baseline: 2.4509x; 1.0300x over previous
import functools

import jax
import jax.numpy as jnp
from jax import lax
from jax.experimental import pallas as pl
from jax.experimental.pallas import tpu as pltpu
from jax.experimental.pallas import tpu_sc as plsc

N = 10000
E = 160000
D_IN = 128
H1 = 128
H2 = 32

NC, NS = 2, 16
NW = NC * NS
CHUNK = 128
CPW = 40
E_PAD = NW * CPW * CHUNK
R_PAD = 10112
RPT = R_PAD // NS

BM = 1024
NBUF = 2
NDST = 4


@functools.lru_cache(maxsize=None)
def _make_sc_agg(with_deg):
    mesh = plsc.VectorSubcoreMesh(core_axis_name="c", subcore_axis_name="s",
                                  num_cores=NC, num_subcores=NS)
    out_type = [jax.ShapeDtypeStruct((NC, R_PAD, H1), jnp.float32)]
    scratch = [
        pltpu.VMEM((CPW, CHUNK), jnp.int32),
        pltpu.VMEM((NDST, CHUNK), jnp.int32),
        pltpu.VMEM((NBUF, CHUNK, H1), jnp.float32),
        pltpu.VMEM_SHARED((R_PAD, H1), jnp.float32),
        pltpu.SemaphoreType.DMA((NBUF,)),
        pltpu.SemaphoreType.DMA((NBUF,)),
        pltpu.SemaphoreType.DMA((NDST,)),
    ]
    if with_deg:
        out_type.append(jax.ShapeDtypeStruct((NW, R_PAD), jnp.float32))
        scratch.append(pltpu.VMEM((R_PAD,), jnp.float32))

    @functools.partial(pl.kernel, out_type=out_type, mesh=mesh,
                       scratch_types=scratch,
                       compiler_params=pltpu.CompilerParams(
                           needs_layout_passes=False))
    def sc_agg(y_hbm, src_hbm, dst_hbm, zeros_hbm, zerosd_hbm, *refs):
        if with_deg:
            (out_hbm, deg_hbm, src_v, dst_r, rows_v, acc_sh,
             gsem, ssem, dsem, deg_v) = refs
        else:
            out_hbm, src_v, dst_r, rows_v, acc_sh, gsem, ssem, dsem = refs
        cid = lax.axis_index("c")
        sid = lax.axis_index("s")
        wid = cid * NS + sid
        with jax.named_scope("sc_zero"):
            pltpu.sync_copy(src_hbm.at[wid], src_v)
            pltpu.sync_copy(zeros_hbm.at[pl.ds(sid * RPT, RPT)],
                            acc_sh.at[pl.ds(sid * RPT, RPT)])
            if with_deg:
                pltpu.sync_copy(zerosd_hbm, deg_v)
            plsc.subcore_barrier()

        ones16 = jnp.full((16,), 1.0, jnp.float32)

        def dst_load(s, j):
            pltpu.async_copy(dst_hbm.at[wid].at[j], dst_r.at[s], dsem.at[s])

        def gather(r, j):
            pltpu.async_copy(y_hbm.at[src_v.at[j]], rows_v.at[r], gsem.at[r])

        for s in range(NDST):
            dst_load(s, s)
        for r in range(NBUF):
            gather(r, r)

        def body(g, _):
            for b in range(NDST):
                j = g * NDST + b
                r = b % NBUF
                s = b
                pltpu.make_async_copy(y_hbm.at[src_v.at[j]], rows_v.at[r],
                                      gsem.at[r]).wait()
                pltpu.make_async_copy(dst_hbm.at[wid].at[j], dst_r.at[s],
                                      dsem.at[s]).wait()
                pltpu.async_copy(rows_v.at[r], acc_sh.at[dst_r.at[s]],
                                 ssem.at[r], add=True)
                if with_deg:
                    for t in range(CHUNK // 16):
                        idx = dst_r[s, pl.ds(t * 16, 16)]
                        plsc.addupdate_scatter(deg_v, [idx], ones16)
                pltpu.make_async_copy(rows_v.at[r], acc_sh.at[dst_r.at[s]],
                                      ssem.at[r]).wait()

                @pl.when(j + NDST < CPW)
                def _():
                    dst_load(s, j + NDST)

                @pl.when(j + NBUF < CPW)
                def _():
                    gather(r, j + NBUF)
            return ()

        with jax.named_scope("sc_mainloop"):
            lax.fori_loop(0, CPW // NDST, body, ())
        with jax.named_scope("sc_bar"):
            plsc.subcore_barrier()
        with jax.named_scope("sc_copyout"):
            pltpu.sync_copy(acc_sh.at[pl.ds(sid * RPT, RPT)],
                            out_hbm.at[cid, pl.ds(sid * RPT, RPT)])
            if with_deg:
                pltpu.sync_copy(deg_v, deg_hbm.at[wid])

    return sc_agg


def _agg(y, src3, dst3, zeros, zerosd, with_deg):
    return _make_sc_agg(with_deg)(y, src3, dst3, zeros, zerosd)


def _proj1_body(x_ref, wn_ref, ws_ref, y_ref, s_ref):
    x = x_ref[...]
    y_ref[...] = jnp.dot(x, wn_ref[...], preferred_element_type=jnp.float32)
    s_ref[...] = jnp.dot(x, ws_ref[...], preferred_element_type=jnp.float32)


def _proj1(x, w_neigh, w_self):
    return pl.pallas_call(
        _proj1_body,
        grid=(pl.cdiv(N, BM),),
        in_specs=[
            pl.BlockSpec((BM, D_IN), lambda i: (i, 0)),
            pl.BlockSpec((D_IN, H1), lambda i: (0, 0)),
            pl.BlockSpec((D_IN, H1), lambda i: (0, 0)),
        ],
        out_specs=[
            pl.BlockSpec((BM, H1), lambda i: (i, 0)),
            pl.BlockSpec((BM, H1), lambda i: (i, 0)),
        ],
        out_shape=[
            jax.ShapeDtypeStruct((N, H1), jnp.float32),
            jax.ShapeDtypeStruct((N, H1), jnp.float32),
        ],
    )(x, w_neigh, w_self)


def _layerB_body(s1_ref, p0_ref, p1_ref, deg_ref, b1_ref, ws_ref,
                 h_ref, s2_ref):
    agg = p0_ref[0] + p1_ref[0]
    deg = jnp.sum(deg_ref[...], axis=0)
    hn = agg / jnp.maximum(deg, 1.0)[:, None]
    h = jnp.maximum(s1_ref[...] + hn + b1_ref[...], 0.0)
    h_ref[...] = h
    s2_ref[...] = jnp.dot(h, ws_ref[...], preferred_element_type=jnp.float32)


def _layerB(s1, p, deg, b1, w_self):
    return pl.pallas_call(
        _layerB_body,
        grid=(pl.cdiv(N, BM),),
        in_specs=[
            pl.BlockSpec((BM, H1), lambda i: (i, 0)),
            pl.BlockSpec((1, BM, H1), lambda i: (0, i, 0)),
            pl.BlockSpec((1, BM, H1), lambda i: (1, i, 0)),
            pl.BlockSpec((NW, BM), lambda i: (0, i)),
            pl.BlockSpec((1, H1), lambda i: (0, 0)),
            pl.BlockSpec((H1, H2), lambda i: (0, 0)),
        ],
        out_specs=[
            pl.BlockSpec((BM, H1), lambda i: (i, 0)),
            pl.BlockSpec((BM, H2), lambda i: (i, 0)),
        ],
        out_shape=[
            jax.ShapeDtypeStruct((N, H1), jnp.float32),
            jax.ShapeDtypeStruct((N, H2), jnp.float32),
        ],
    )(s1, p, p, deg, b1, w_self)


def _layerC1_body(s2_ref, q0_ref, q1_ref, deg_ref, b2_ref, wn_ref, z_ref):
    agg = q0_ref[0] + q1_ref[0]
    deg = jnp.sum(deg_ref[...], axis=0)
    hn = agg / jnp.maximum(deg, 1.0)[:, None]
    proj = jnp.dot(hn, wn_ref[...], preferred_element_type=jnp.float32)
    z_ref[...] = jnp.maximum(s2_ref[...] + proj + b2_ref[...], 0.0)


def _layerC1(s2, q, deg, b2, w_neigh):
    return pl.pallas_call(
        _layerC1_body,
        grid=(pl.cdiv(N, BM),),
        in_specs=[
            pl.BlockSpec((BM, H2), lambda i: (i, 0)),
            pl.BlockSpec((1, BM, H1), lambda i: (0, i, 0)),
            pl.BlockSpec((1, BM, H1), lambda i: (1, i, 0)),
            pl.BlockSpec((NW, BM), lambda i: (0, i)),
            pl.BlockSpec((1, H2), lambda i: (0, 0)),
            pl.BlockSpec((H1, H2), lambda i: (0, 0)),
        ],
        out_specs=pl.BlockSpec((BM, H2), lambda i: (i, 0)),
        out_shape=jax.ShapeDtypeStruct((N, H2), jnp.float32),
    )(s2, q, q, deg, b2, w_neigh)


def _dec_body(zi_ref, zj_ref, out_ref):
    prod = lax.dot_general(zi_ref[...], zj_ref[...],
                           (((1,), (1,)), ((), ())),
                           preferred_element_type=jnp.float32)
    out_ref[...] = 0.5 * jnp.tanh(0.5 * prod) + 0.5


DEC_BM = 2048
DEC_BN = 2048


def _decoder(z):
    return pl.pallas_call(
        _dec_body,
        grid=(pl.cdiv(N, DEC_BM), pl.cdiv(N, DEC_BN)),
        in_specs=[
            pl.BlockSpec((DEC_BM, H2), lambda i, j: (i, 0)),
            pl.BlockSpec((DEC_BN, H2), lambda i, j: (j, 0)),
        ],
        out_specs=pl.BlockSpec((DEC_BM, DEC_BN), lambda i, j: (i, j)),
        out_shape=jax.ShapeDtypeStruct((N, N), jnp.float32),
    )(z, z)


def kernel(features, edge_index, W1_self, W1_neigh, b1, W2_self, W2_neigh, b2):
    src = edge_index[0].astype(jnp.int32)
    dst = edge_index[1].astype(jnp.int32)
    pad_ar = jnp.arange(E_PAD - E, dtype=jnp.int32)
    pad_dst = N + pad_ar % (R_PAD - N)
    pad_src = pad_ar % N
    src3 = jnp.concatenate([src, pad_src]).reshape(NW, CPW, CHUNK)
    dst3 = jnp.concatenate([dst, pad_dst]).reshape(NW, CPW, CHUNK)
    zeros2d = jnp.zeros((R_PAD, H1), jnp.float32)
    zeros1d = jnp.zeros((R_PAD,), jnp.float32)

    y1, s1 = _proj1(features, W1_neigh, W1_self)
    p, deg = _agg(y1, src3, dst3, zeros2d, zeros1d, True)
    h, s2 = _layerB(s1, p, deg, b1.reshape(1, H1), W2_self)
    (q,) = _agg(h, src3, dst3, zeros2d, zeros1d, False)
    z = _layerC1(s2, q, deg, b2.reshape(1, H2), W2_neigh)
    return _decoder(z)


# speedup vs baseline: 2.4843x; 1.0136x over previous
import functools

import jax
import jax.numpy as jnp
from jax import lax
from jax.experimental import pallas as pl
from jax.experimental.pallas import tpu as pltpu
from jax.experimental.pallas import tpu_sc as plsc

N = 10000
E = 160000
D_IN = 128
H1 = 128
H2 = 32

NC, NS = 2, 16
NW = NC * NS
CHUNK = 128
CPW = 40
E_PAD = NW * CPW * CHUNK
R_PAD = 10112
RPT = R_PAD // NS

BM = 1024
NBUF = 2
NDST = 4


@functools.lru_cache(maxsize=None)
def _make_sc_agg(with_deg):
    mesh = plsc.VectorSubcoreMesh(core_axis_name="c", subcore_axis_name="s",
                                  num_cores=NC, num_subcores=NS)
    out_type = [jax.ShapeDtypeStruct((NC, R_PAD, H1), jnp.float32)]
    scratch = [
        pltpu.VMEM((CPW, CHUNK), jnp.int32),
        pltpu.VMEM((NDST, CHUNK), jnp.int32),
        pltpu.VMEM((NBUF, CHUNK, H1), jnp.float32),
        pltpu.VMEM_SHARED((R_PAD, H1), jnp.float32),
        pltpu.SemaphoreType.DMA((NBUF,)),
        pltpu.SemaphoreType.DMA((NBUF,)),
        pltpu.SemaphoreType.DMA((NDST,)),
        pltpu.SemaphoreType.DMA,
    ]
    if with_deg:
        out_type.append(jax.ShapeDtypeStruct((NW, R_PAD), jnp.float32))
        scratch.append(pltpu.VMEM((R_PAD,), jnp.float32))

    @functools.partial(pl.kernel, out_type=out_type, mesh=mesh,
                       scratch_types=scratch,
                       compiler_params=pltpu.CompilerParams(
                           needs_layout_passes=False))
    def sc_agg(y_hbm, src_hbm, dst_hbm, zeros_hbm, zerosd_hbm, *refs):
        if with_deg:
            (out_hbm, deg_hbm, src_v, dst_r, rows_v, acc_sh,
             gsem, ssem, dsem, zsem, deg_v) = refs
        else:
            (out_hbm, src_v, dst_r, rows_v, acc_sh,
             gsem, ssem, dsem, zsem) = refs
        cid = lax.axis_index("c")
        sid = lax.axis_index("s")
        wid = cid * NS + sid

        ones16 = jnp.full((16,), 1.0, jnp.float32)

        def dst_load(s, j):
            pltpu.async_copy(dst_hbm.at[wid].at[j], dst_r.at[s], dsem.at[s])

        def gather(r, j):
            pltpu.async_copy(y_hbm.at[src_v.at[j]], rows_v.at[r], gsem.at[r])

        with jax.named_scope("sc_zero"):
            zero_cp = pltpu.async_copy(zeros_hbm.at[pl.ds(sid * RPT, RPT)],
                                       acc_sh.at[pl.ds(sid * RPT, RPT)], zsem)
            pltpu.sync_copy(src_hbm.at[wid], src_v)
            for s in range(NDST):
                dst_load(s, s)
            for r in range(NBUF):
                gather(r, r)
            if with_deg:
                pltpu.sync_copy(zerosd_hbm, deg_v)
            zero_cp.wait()
            plsc.subcore_barrier()

        def body(g, _):
            for b in range(NDST):
                j = g * NDST + b
                r = b % NBUF
                s = b
                pltpu.make_async_copy(y_hbm.at[src_v.at[j]], rows_v.at[r],
                                      gsem.at[r]).wait()
                pltpu.make_async_copy(dst_hbm.at[wid].at[j], dst_r.at[s],
                                      dsem.at[s]).wait()
                pltpu.async_copy(rows_v.at[r], acc_sh.at[dst_r.at[s]],
                                 ssem.at[r], add=True)
                if with_deg:
                    for t in range(CHUNK // 16):
                        idx = dst_r[s, pl.ds(t * 16, 16)]
                        plsc.addupdate_scatter(deg_v, [idx], ones16)
                pltpu.make_async_copy(rows_v.at[r], acc_sh.at[dst_r.at[s]],
                                      ssem.at[r]).wait()

                @pl.when(j + NDST < CPW)
                def _():
                    dst_load(s, j + NDST)

                @pl.when(j + NBUF < CPW)
                def _():
                    gather(r, j + NBUF)
            return ()

        with jax.named_scope("sc_mainloop"):
            lax.fori_loop(0, CPW // NDST, body, ())
        with jax.named_scope("sc_bar"):
            plsc.subcore_barrier()
        with jax.named_scope("sc_copyout"):
            pltpu.sync_copy(acc_sh.at[pl.ds(sid * RPT, RPT)],
                            out_hbm.at[cid, pl.ds(sid * RPT, RPT)])
            if with_deg:
                pltpu.sync_copy(deg_v, deg_hbm.at[wid])

    return sc_agg


def _agg(y, src3, dst3, zeros, zerosd, with_deg):
    return _make_sc_agg(with_deg)(y, src3, dst3, zeros, zerosd)


def _proj1_body(x_ref, wn_ref, ws_ref, y_ref, s_ref):
    x = x_ref[...]
    y_ref[...] = jnp.dot(x, wn_ref[...], preferred_element_type=jnp.float32)
    s_ref[...] = jnp.dot(x, ws_ref[...], preferred_element_type=jnp.float32)


def _proj1(x, w_neigh, w_self):
    return pl.pallas_call(
        _proj1_body,
        grid=(pl.cdiv(N, BM),),
        in_specs=[
            pl.BlockSpec((BM, D_IN), lambda i: (i, 0)),
            pl.BlockSpec((D_IN, H1), lambda i: (0, 0)),
            pl.BlockSpec((D_IN, H1), lambda i: (0, 0)),
        ],
        out_specs=[
            pl.BlockSpec((BM, H1), lambda i: (i, 0)),
            pl.BlockSpec((BM, H1), lambda i: (i, 0)),
        ],
        out_shape=[
            jax.ShapeDtypeStruct((N, H1), jnp.float32),
            jax.ShapeDtypeStruct((N, H1), jnp.float32),
        ],
    )(x, w_neigh, w_self)


def _layerB_body(s1_ref, p0_ref, p1_ref, deg_ref, b1_ref, ws_ref,
                 h_ref, s2_ref):
    agg = p0_ref[0] + p1_ref[0]
    deg = jnp.sum(deg_ref[...], axis=0)
    hn = agg / jnp.maximum(deg, 1.0)[:, None]
    h = jnp.maximum(s1_ref[...] + hn + b1_ref[...], 0.0)
    h_ref[...] = h
    s2_ref[...] = jnp.dot(h, ws_ref[...], preferred_element_type=jnp.float32)


def _layerB(s1, p, deg, b1, w_self):
    return pl.pallas_call(
        _layerB_body,
        grid=(pl.cdiv(N, BM),),
        in_specs=[
            pl.BlockSpec((BM, H1), lambda i: (i, 0)),
            pl.BlockSpec((1, BM, H1), lambda i: (0, i, 0)),
            pl.BlockSpec((1, BM, H1), lambda i: (1, i, 0)),
            pl.BlockSpec((NW, BM), lambda i: (0, i)),
            pl.BlockSpec((1, H1), lambda i: (0, 0)),
            pl.BlockSpec((H1, H2), lambda i: (0, 0)),
        ],
        out_specs=[
            pl.BlockSpec((BM, H1), lambda i: (i, 0)),
            pl.BlockSpec((BM, H2), lambda i: (i, 0)),
        ],
        out_shape=[
            jax.ShapeDtypeStruct((N, H1), jnp.float32),
            jax.ShapeDtypeStruct((N, H2), jnp.float32),
        ],
    )(s1, p, p, deg, b1, w_self)


def _layerC1_body(s2_ref, q0_ref, q1_ref, deg_ref, b2_ref, wn_ref, z_ref):
    agg = q0_ref[0] + q1_ref[0]
    deg = jnp.sum(deg_ref[...], axis=0)
    hn = agg / jnp.maximum(deg, 1.0)[:, None]
    proj = jnp.dot(hn, wn_ref[...], preferred_element_type=jnp.float32)
    z_ref[...] = jnp.maximum(s2_ref[...] + proj + b2_ref[...], 0.0)


def _layerC1(s2, q, deg, b2, w_neigh):
    return pl.pallas_call(
        _layerC1_body,
        grid=(pl.cdiv(N, BM),),
        in_specs=[
            pl.BlockSpec((BM, H2), lambda i: (i, 0)),
            pl.BlockSpec((1, BM, H1), lambda i: (0, i, 0)),
            pl.BlockSpec((1, BM, H1), lambda i: (1, i, 0)),
            pl.BlockSpec((NW, BM), lambda i: (0, i)),
            pl.BlockSpec((1, H2), lambda i: (0, 0)),
            pl.BlockSpec((H1, H2), lambda i: (0, 0)),
        ],
        out_specs=pl.BlockSpec((BM, H2), lambda i: (i, 0)),
        out_shape=jax.ShapeDtypeStruct((N, H2), jnp.float32),
    )(s2, q, q, deg, b2, w_neigh)


def _dec_body(zi_ref, zj_ref, out_ref):
    prod = lax.dot_general(zi_ref[...], zj_ref[...],
                           (((1,), (1,)), ((), ())),
                           preferred_element_type=jnp.float32)
    out_ref[...] = 0.5 * jnp.tanh(0.5 * prod) + 0.5


DEC_BM = 2048
DEC_BN = 2048


def _decoder(z):
    return pl.pallas_call(
        _dec_body,
        grid=(pl.cdiv(N, DEC_BM), pl.cdiv(N, DEC_BN)),
        in_specs=[
            pl.BlockSpec((DEC_BM, H2), lambda i, j: (i, 0)),
            pl.BlockSpec((DEC_BN, H2), lambda i, j: (j, 0)),
        ],
        out_specs=pl.BlockSpec((DEC_BM, DEC_BN), lambda i, j: (i, j)),
        out_shape=jax.ShapeDtypeStruct((N, N), jnp.float32),
    )(z, z)


def kernel(features, edge_index, W1_self, W1_neigh, b1, W2_self, W2_neigh, b2):
    src = edge_index[0].astype(jnp.int32)
    dst = edge_index[1].astype(jnp.int32)
    pad_ar = jnp.arange(E_PAD - E, dtype=jnp.int32)
    pad_dst = N + pad_ar % (R_PAD - N)
    pad_src = pad_ar % N
    src3 = jnp.concatenate([src, pad_src]).reshape(NW, CPW, CHUNK)
    dst3 = jnp.concatenate([dst, pad_dst]).reshape(NW, CPW, CHUNK)
    zeros2d = jnp.zeros((R_PAD, H1), jnp.float32)
    zeros1d = jnp.zeros((R_PAD,), jnp.float32)

    y1, s1 = _proj1(features, W1_neigh, W1_self)
    p, deg = _agg(y1, src3, dst3, zeros2d, zeros1d, True)
    h, s2 = _layerB(s1, p, deg, b1.reshape(1, H1), W2_self)
    (q,) = _agg(h, src3, dst3, zeros2d, zeros1d, False)
    z = _layerC1(s2, q, deg, b2.reshape(1, H2), W2_neigh)
    return _decoder(z)


# speedup vs baseline: 2.5205x; 1.0146x over previous
import functools

import jax
import jax.numpy as jnp
from jax import lax
from jax.experimental import pallas as pl
from jax.experimental.pallas import tpu as pltpu
from jax.experimental.pallas import tpu_sc as plsc

N = 10000
E = 160000
D_IN = 128
H1 = 128
H2 = 32

NC, NS = 2, 16
NW = NC * NS
CHUNK = 128
CPW = 40
E_PAD = NW * CPW * CHUNK
R_PAD = 10112
RPT = R_PAD // NS

BM = 1024
NBUF = 2
NDST = 4


@functools.lru_cache(maxsize=None)
def _make_sc_agg(with_deg):
    mesh = plsc.VectorSubcoreMesh(core_axis_name="c", subcore_axis_name="s",
                                  num_cores=NC, num_subcores=NS)
    out_type = [jax.ShapeDtypeStruct((NC, R_PAD, H1), jnp.float32)]
    scratch = [
        pltpu.VMEM((CPW, CHUNK), jnp.int32),
        pltpu.VMEM((NDST, CHUNK), jnp.int32),
        pltpu.VMEM((NBUF, CHUNK, H1), jnp.float32),
        pltpu.VMEM_SHARED((R_PAD, H1), jnp.float32),
        pltpu.SemaphoreType.DMA((NBUF,)),
        pltpu.SemaphoreType.DMA((NBUF,)),
        pltpu.SemaphoreType.DMA((NDST,)),
        pltpu.SemaphoreType.DMA,
    ]
    if with_deg:
        out_type.append(jax.ShapeDtypeStruct((NW, R_PAD), jnp.float32))
        scratch.append(pltpu.VMEM((R_PAD,), jnp.float32))

    @functools.partial(pl.kernel, out_type=out_type, mesh=mesh,
                       scratch_types=scratch,
                       compiler_params=pltpu.CompilerParams(
                           needs_layout_passes=False))
    def sc_agg(y_hbm, src_hbm, dst_hbm, zeros_hbm, zerosd_hbm, *refs):
        if with_deg:
            (out_hbm, deg_hbm, src_v, dst_r, rows_v, acc_sh,
             gsem, ssem, dsem, zsem, deg_v) = refs
        else:
            (out_hbm, src_v, dst_r, rows_v, acc_sh,
             gsem, ssem, dsem, zsem) = refs
        cid = lax.axis_index("c")
        sid = lax.axis_index("s")
        wid = cid * NS + sid

        ones16 = jnp.full((16,), 1.0, jnp.float32)

        def dst_load(s, j):
            pltpu.async_copy(dst_hbm.at[wid].at[j], dst_r.at[s], dsem.at[s])

        def gather(r, j):
            pltpu.async_copy(y_hbm.at[src_v.at[j]], rows_v.at[r], gsem.at[r])

        with jax.named_scope("sc_zero"):
            zero_cp = pltpu.async_copy(zeros_hbm.at[pl.ds(sid * RPT, RPT)],
                                       acc_sh.at[pl.ds(sid * RPT, RPT)], zsem)
            pltpu.sync_copy(src_hbm.at[wid], src_v)
            for s in range(NDST):
                dst_load(s, s)
            for r in range(NBUF):
                gather(r, r)
            if with_deg:
                pltpu.sync_copy(zerosd_hbm, deg_v)
            zero_cp.wait()
            plsc.subcore_barrier()

        def body(g, _):
            for b in range(NDST):
                j = g * NDST + b
                r = b % NBUF
                s = b
                pltpu.make_async_copy(y_hbm.at[src_v.at[j]], rows_v.at[r],
                                      gsem.at[r]).wait()
                pltpu.make_async_copy(dst_hbm.at[wid].at[j], dst_r.at[s],
                                      dsem.at[s]).wait()
                pltpu.async_copy(rows_v.at[r], acc_sh.at[dst_r.at[s]],
                                 ssem.at[r], add=True)
                if with_deg:
                    for t in range(CHUNK // 16):
                        idx = dst_r[s, pl.ds(t * 16, 16)]
                        plsc.addupdate_scatter(deg_v, [idx], ones16)
                pltpu.make_async_copy(rows_v.at[r], acc_sh.at[dst_r.at[s]],
                                      ssem.at[r]).wait()

                @pl.when(j + NDST < CPW)
                def _():
                    dst_load(s, j + NDST)

                @pl.when(j + NBUF < CPW)
                def _():
                    gather(r, j + NBUF)
            return ()

        with jax.named_scope("sc_mainloop"):
            lax.fori_loop(0, CPW // NDST, body, ())
        with jax.named_scope("sc_bar"):
            plsc.subcore_barrier()
        with jax.named_scope("sc_copyout"):
            pltpu.sync_copy(acc_sh.at[pl.ds(sid * RPT, RPT)],
                            out_hbm.at[cid, pl.ds(sid * RPT, RPT)])
            if with_deg:
                pltpu.sync_copy(deg_v, deg_hbm.at[wid])

    return sc_agg


def _agg(y, src3, dst3, zeros, zerosd, with_deg):
    return _make_sc_agg(with_deg)(y, src3, dst3, zeros, zerosd)


def _proj1_body(x_ref, wn_ref, ws_ref, y_ref, s_ref):
    x = x_ref[...]
    y_ref[...] = jnp.dot(x, wn_ref[...], preferred_element_type=jnp.float32)
    s_ref[...] = jnp.dot(x, ws_ref[...], preferred_element_type=jnp.float32)


def _proj1(x, w_neigh, w_self):
    return pl.pallas_call(
        _proj1_body,
        grid=(pl.cdiv(N, BM),),
        in_specs=[
            pl.BlockSpec((BM, D_IN), lambda i: (i, 0)),
            pl.BlockSpec((D_IN, H1), lambda i: (0, 0)),
            pl.BlockSpec((D_IN, H1), lambda i: (0, 0)),
        ],
        out_specs=[
            pl.BlockSpec((BM, H1), lambda i: (i, 0)),
            pl.BlockSpec((BM, H1), lambda i: (i, 0)),
        ],
        out_shape=[
            jax.ShapeDtypeStruct((N, H1), jnp.float32),
            jax.ShapeDtypeStruct((N, H1), jnp.float32),
        ],
    )(x, w_neigh, w_self)


def _layerB_body(s1_ref, p0_ref, p1_ref, deg_ref, b1_ref, ws_ref,
                 h_ref, s2_ref):
    agg = p0_ref[0] + p1_ref[0]
    deg = jnp.sum(deg_ref[...], axis=0)
    hn = agg / jnp.maximum(deg, 1.0)[:, None]
    h = jnp.maximum(s1_ref[...] + hn + b1_ref[...], 0.0)
    h_ref[...] = h
    s2_ref[...] = jnp.dot(h, ws_ref[...], preferred_element_type=jnp.float32)


def _layerB(s1, p, deg, b1, w_self):
    return pl.pallas_call(
        _layerB_body,
        grid=(pl.cdiv(N, BM),),
        in_specs=[
            pl.BlockSpec((BM, H1), lambda i: (i, 0)),
            pl.BlockSpec((1, BM, H1), lambda i: (0, i, 0)),
            pl.BlockSpec((1, BM, H1), lambda i: (1, i, 0)),
            pl.BlockSpec((NW, BM), lambda i: (0, i)),
            pl.BlockSpec((1, H1), lambda i: (0, 0)),
            pl.BlockSpec((H1, H2), lambda i: (0, 0)),
        ],
        out_specs=[
            pl.BlockSpec((BM, H1), lambda i: (i, 0)),
            pl.BlockSpec((BM, H2), lambda i: (i, 0)),
        ],
        out_shape=[
            jax.ShapeDtypeStruct((N, H1), jnp.float32),
            jax.ShapeDtypeStruct((N, H2), jnp.float32),
        ],
    )(s1, p, p, deg, b1, w_self)


def _layerC1_body(s2_ref, q0_ref, q1_ref, deg_ref, b2_ref, wn_ref, z_ref):
    agg = q0_ref[0] + q1_ref[0]
    deg = jnp.sum(deg_ref[...], axis=0)
    hn = agg / jnp.maximum(deg, 1.0)[:, None]
    proj = jnp.dot(hn, wn_ref[...], preferred_element_type=jnp.float32)
    z_ref[...] = jnp.maximum(s2_ref[...] + proj + b2_ref[...], 0.0)


def _layerC1(s2, q, deg, b2, w_neigh):
    return pl.pallas_call(
        _layerC1_body,
        grid=(pl.cdiv(N, BM),),
        in_specs=[
            pl.BlockSpec((BM, H2), lambda i: (i, 0)),
            pl.BlockSpec((1, BM, H1), lambda i: (0, i, 0)),
            pl.BlockSpec((1, BM, H1), lambda i: (1, i, 0)),
            pl.BlockSpec((NW, BM), lambda i: (0, i)),
            pl.BlockSpec((1, H2), lambda i: (0, 0)),
            pl.BlockSpec((H1, H2), lambda i: (0, 0)),
        ],
        out_specs=pl.BlockSpec((BM, H2), lambda i: (i, 0)),
        out_shape=jax.ShapeDtypeStruct((N, H2), jnp.float32),
    )(s2, q, q, deg, b2, w_neigh)


def _dec_body(zi_ref, zj_ref, out_ref):
    prod = lax.dot_general(zi_ref[...], zj_ref[...],
                           (((1,), (1,)), ((), ())),
                           preferred_element_type=jnp.float32)
    out_ref[...] = 0.5 * jnp.tanh(0.5 * prod) + 0.5


DEC_BM = 512


def _decoder(z):
    return pl.pallas_call(
        _dec_body,
        grid=(pl.cdiv(N, DEC_BM),),
        in_specs=[
            pl.BlockSpec((DEC_BM, H2), lambda i: (i, 0)),
            pl.BlockSpec((N, H2), lambda i: (0, 0)),
        ],
        out_specs=pl.BlockSpec((DEC_BM, N), lambda i: (i, 0)),
        out_shape=jax.ShapeDtypeStruct((N, N), jnp.float32),
    )(z, z)


def kernel(features, edge_index, W1_self, W1_neigh, b1, W2_self, W2_neigh, b2):
    src = edge_index[0].astype(jnp.int32)
    dst = edge_index[1].astype(jnp.int32)
    pad_ar = jnp.arange(E_PAD - E, dtype=jnp.int32)
    pad_dst = N + pad_ar % (R_PAD - N)
    pad_src = pad_ar % N
    src3 = jnp.concatenate([src, pad_src]).reshape(NW, CPW, CHUNK)
    dst3 = jnp.concatenate([dst, pad_dst]).reshape(NW, CPW, CHUNK)
    zeros2d = jnp.zeros((R_PAD, H1), jnp.float32)
    zeros1d = jnp.zeros((R_PAD,), jnp.float32)

    y1, s1 = _proj1(features, W1_neigh, W1_self)
    p, deg = _agg(y1, src3, dst3, zeros2d, zeros1d, True)
    h, s2 = _layerB(s1, p, deg, b1.reshape(1, H1), W2_self)
    (q,) = _agg(h, src3, dst3, zeros2d, zeros1d, False)
    z = _layerC1(s2, q, deg, b2.reshape(1, H2), W2_neigh)
    return _decoder(z)


# speedup vs baseline: 2.5345x; 1.0056x over previous
import functools

import jax
import jax.numpy as jnp
from jax import lax
from jax.experimental import pallas as pl
from jax.experimental.pallas import tpu as pltpu
from jax.experimental.pallas import tpu_sc as plsc

N = 10000
E = 160000
D_IN = 128
H1 = 128
H2 = 32

NC, NS = 2, 16
NW = NC * NS
CHUNK = 128
CPW = 40
E_PAD = NW * CPW * CHUNK
R_PAD = 10112
RPT = R_PAD // NS

BM = 1024
NBUF = 2
NDST = 4


@functools.lru_cache(maxsize=None)
def _make_sc_agg(with_deg):
    mesh = plsc.VectorSubcoreMesh(core_axis_name="c", subcore_axis_name="s",
                                  num_cores=NC, num_subcores=NS)
    out_type = [jax.ShapeDtypeStruct((NC, R_PAD, H1), jnp.float32)]
    scratch = [
        pltpu.VMEM((CPW, CHUNK), jnp.int32),
        pltpu.VMEM((NDST, CHUNK), jnp.int32),
        pltpu.VMEM((NBUF, CHUNK, H1), jnp.float32),
        pltpu.VMEM_SHARED((R_PAD, H1), jnp.float32),
        pltpu.SemaphoreType.DMA((NBUF,)),
        pltpu.SemaphoreType.DMA((NBUF,)),
        pltpu.SemaphoreType.DMA((NDST,)),
        pltpu.SemaphoreType.DMA,
    ]
    if with_deg:
        out_type.append(jax.ShapeDtypeStruct((NW, R_PAD), jnp.float32))
        scratch.append(pltpu.VMEM((R_PAD,), jnp.float32))

    @functools.partial(pl.kernel, out_type=out_type, mesh=mesh,
                       scratch_types=scratch,
                       compiler_params=pltpu.CompilerParams(
                           needs_layout_passes=False))
    def sc_agg(y_hbm, src_hbm, dst_hbm, zeros_hbm, zerosd_hbm, *refs):
        if with_deg:
            (out_hbm, deg_hbm, src_v, dst_r, rows_v, acc_sh,
             gsem, ssem, dsem, zsem, deg_v) = refs
        else:
            (out_hbm, src_v, dst_r, rows_v, acc_sh,
             gsem, ssem, dsem, zsem) = refs
        cid = lax.axis_index("c")
        sid = lax.axis_index("s")
        wid = cid * NS + sid

        ones16 = jnp.full((16,), 1.0, jnp.float32)

        def dst_load(s, j):
            pltpu.async_copy(dst_hbm.at[wid].at[j], dst_r.at[s], dsem.at[s])

        def gather(r, j):
            pltpu.async_copy(y_hbm.at[src_v.at[j]], rows_v.at[r], gsem.at[r])

        with jax.named_scope("sc_zero"):
            zero_cp = pltpu.async_copy(zeros_hbm.at[pl.ds(sid * RPT, RPT)],
                                       acc_sh.at[pl.ds(sid * RPT, RPT)], zsem)
            pltpu.sync_copy(src_hbm.at[wid], src_v)
            for s in range(NDST):
                dst_load(s, s)
            for r in range(NBUF):
                gather(r, r)
            if with_deg:
                pltpu.sync_copy(zerosd_hbm, deg_v)
            zero_cp.wait()
            plsc.subcore_barrier()

        def body(g, _):
            for b in range(NDST):
                j = g * NDST + b
                r = b % NBUF
                s = b
                pltpu.make_async_copy(y_hbm.at[src_v.at[j]], rows_v.at[r],
                                      gsem.at[r]).wait()
                pltpu.make_async_copy(dst_hbm.at[wid].at[j], dst_r.at[s],
                                      dsem.at[s]).wait()
                pltpu.async_copy(rows_v.at[r], acc_sh.at[dst_r.at[s]],
                                 ssem.at[r], add=True)
                if with_deg:
                    for t in range(CHUNK // 16):
                        idx = dst_r[s, pl.ds(t * 16, 16)]
                        plsc.addupdate_scatter(deg_v, [idx], ones16)
                pltpu.make_async_copy(rows_v.at[r], acc_sh.at[dst_r.at[s]],
                                      ssem.at[r]).wait()

                @pl.when(j + NDST < CPW)
                def _():
                    dst_load(s, j + NDST)

                @pl.when(j + NBUF < CPW)
                def _():
                    gather(r, j + NBUF)
            return ()

        with jax.named_scope("sc_mainloop"):
            lax.fori_loop(0, CPW // NDST, body, ())
        with jax.named_scope("sc_bar"):
            plsc.subcore_barrier()
        with jax.named_scope("sc_copyout"):
            pltpu.sync_copy(acc_sh.at[pl.ds(sid * RPT, RPT)],
                            out_hbm.at[cid, pl.ds(sid * RPT, RPT)])
            if with_deg:
                pltpu.sync_copy(deg_v, deg_hbm.at[wid])

    return sc_agg


def _agg(y, src3, dst3, zeros, zerosd, with_deg):
    return _make_sc_agg(with_deg)(y, src3, dst3, zeros, zerosd)


def _proj1_body(x_ref, wn_ref, ws_ref, y_ref, s_ref):
    x = x_ref[...]
    y_ref[...] = jnp.dot(x, wn_ref[...], preferred_element_type=jnp.float32)
    s_ref[...] = jnp.dot(x, ws_ref[...], preferred_element_type=jnp.float32)


def _proj1(x, w_neigh, w_self):
    return pl.pallas_call(
        _proj1_body,
        grid=(pl.cdiv(N, BM),),
        in_specs=[
            pl.BlockSpec((BM, D_IN), lambda i: (i, 0)),
            pl.BlockSpec((D_IN, H1), lambda i: (0, 0)),
            pl.BlockSpec((D_IN, H1), lambda i: (0, 0)),
        ],
        out_specs=[
            pl.BlockSpec((BM, H1), lambda i: (i, 0)),
            pl.BlockSpec((BM, H1), lambda i: (i, 0)),
        ],
        out_shape=[
            jax.ShapeDtypeStruct((N, H1), jnp.float32),
            jax.ShapeDtypeStruct((N, H1), jnp.float32),
        ],
    )(x, w_neigh, w_self)


def _layerB_body(s1_ref, p0_ref, p1_ref, deg_ref, b1_ref, ws_ref,
                 h_ref, s2_ref):
    agg = p0_ref[0] + p1_ref[0]
    deg = jnp.sum(deg_ref[...], axis=0)
    hn = agg / jnp.maximum(deg, 1.0)[:, None]
    h = jnp.maximum(s1_ref[...] + hn + b1_ref[...], 0.0)
    h_ref[...] = h
    s2_ref[...] = jnp.dot(h, ws_ref[...], preferred_element_type=jnp.float32)


def _layerB(s1, p, deg, b1, w_self):
    return pl.pallas_call(
        _layerB_body,
        grid=(pl.cdiv(N, BM),),
        in_specs=[
            pl.BlockSpec((BM, H1), lambda i: (i, 0)),
            pl.BlockSpec((1, BM, H1), lambda i: (0, i, 0)),
            pl.BlockSpec((1, BM, H1), lambda i: (1, i, 0)),
            pl.BlockSpec((NW, BM), lambda i: (0, i)),
            pl.BlockSpec((1, H1), lambda i: (0, 0)),
            pl.BlockSpec((H1, H2), lambda i: (0, 0)),
        ],
        out_specs=[
            pl.BlockSpec((BM, H1), lambda i: (i, 0)),
            pl.BlockSpec((BM, H2), lambda i: (i, 0)),
        ],
        out_shape=[
            jax.ShapeDtypeStruct((N, H1), jnp.float32),
            jax.ShapeDtypeStruct((N, H2), jnp.float32),
        ],
    )(s1, p, p, deg, b1, w_self)


def _layerC1_body(s2_ref, q0_ref, q1_ref, deg_ref, b2_ref, wn_ref, z_ref):
    agg = q0_ref[0] + q1_ref[0]
    deg = jnp.sum(deg_ref[...], axis=0)
    hn = agg / jnp.maximum(deg, 1.0)[:, None]
    proj = jnp.dot(hn, wn_ref[...], preferred_element_type=jnp.float32)
    z_ref[...] = jnp.maximum(s2_ref[...] + proj + b2_ref[...], 0.0)


def _layerC1(s2, q, deg, b2, w_neigh):
    return pl.pallas_call(
        _layerC1_body,
        grid=(pl.cdiv(N, BM),),
        in_specs=[
            pl.BlockSpec((BM, H2), lambda i: (i, 0)),
            pl.BlockSpec((1, BM, H1), lambda i: (0, i, 0)),
            pl.BlockSpec((1, BM, H1), lambda i: (1, i, 0)),
            pl.BlockSpec((NW, BM), lambda i: (0, i)),
            pl.BlockSpec((1, H2), lambda i: (0, 0)),
            pl.BlockSpec((H1, H2), lambda i: (0, 0)),
        ],
        out_specs=pl.BlockSpec((BM, H2), lambda i: (i, 0)),
        out_shape=jax.ShapeDtypeStruct((N, H2), jnp.float32),
    )(s2, q, q, deg, b2, w_neigh)


def _dec_body(zi_ref, zj_ref, out_ref):
    prod = lax.dot_general(zi_ref[...], zj_ref[...],
                           (((1,), (1,)), ((), ())),
                           preferred_element_type=jnp.float32)
    out_ref[...] = 0.5 * jnp.tanh(0.5 * prod) + 0.5


DEC_BM = 256


def _decoder(z):
    return pl.pallas_call(
        _dec_body,
        grid=(pl.cdiv(N, DEC_BM),),
        in_specs=[
            pl.BlockSpec((DEC_BM, H2), lambda i: (i, 0)),
            pl.BlockSpec((N, H2), lambda i: (0, 0)),
        ],
        out_specs=pl.BlockSpec((DEC_BM, N), lambda i: (i, 0)),
        out_shape=jax.ShapeDtypeStruct((N, N), jnp.float32),
    )(z, z)


def kernel(features, edge_index, W1_self, W1_neigh, b1, W2_self, W2_neigh, b2):
    src = edge_index[0].astype(jnp.int32)
    dst = edge_index[1].astype(jnp.int32)
    pad_ar = jnp.arange(E_PAD - E, dtype=jnp.int32)
    pad_dst = N + pad_ar % (R_PAD - N)
    pad_src = pad_ar % N
    src3 = jnp.concatenate([src, pad_src]).reshape(NW, CPW, CHUNK)
    dst3 = jnp.concatenate([dst, pad_dst]).reshape(NW, CPW, CHUNK)
    zeros2d = jnp.zeros((R_PAD, H1), jnp.float32)
    zeros1d = jnp.zeros((R_PAD,), jnp.float32)

    y1, s1 = _proj1(features, W1_neigh, W1_self)
    p, deg = _agg(y1, src3, dst3, zeros2d, zeros1d, True)
    h, s2 = _layerB(s1, p, deg, b1.reshape(1, H1), W2_self)
    (q,) = _agg(h, src3, dst3, zeros2d, zeros1d, False)
    z = _layerC1(s2, q, deg, b2.reshape(1, H2), W2_neigh)
    return _decoder(z)
